# Initial kernel scaffold; baseline (speedup 1.0000x reference)
import jax
import jax.numpy as jnp
from jax.experimental import pallas as pl


def kernel(x, edge_index, W1, b1, W2, b2):
    raise NotImplementedError("write your pallas kernel here")


# speedup vs baseline: 13.3257x; 13.3257x over previous
import functools

import jax
import jax.numpy as jnp
from jax import lax
from jax.experimental import pallas as pl
from jax.experimental.pallas import tpu as pltpu
from jax.experimental.pallas import tpu_sc as plsc

N = 10000
E = 320000
D_IN = 128
D_HID = 100
DP = 112
NCLS = 10
CP = 16

NC = 2
NS = 16
NW = NC * NS
EPW = 10240
EPAD = NW * EPW
BLK = 128
NBLK = EPW // BLK
NPAD = 10240
RPW = NPAD // NS

ROWBLK = 512
NROWBLK = NPAD // ROWBLK

_MESH = plsc.VectorSubcoreMesh(core_axis_name="c", subcore_axis_name="s")
_SC_PARAMS = pltpu.CompilerParams(use_tc_tiling_on_sc=False)


def _deg_sc(dstp, ones16, zeros16):
    @functools.partial(
        pl.kernel,
        mesh=_MESH,
        compiler_params=_SC_PARAMS,
        out_type=jax.ShapeDtypeStruct((NC, NPAD, 16), jnp.float32),
        scratch_types=[
            pltpu.VMEM_SHARED((NPAD, 16), jnp.float32),
            pltpu.VMEM((BLK, 16), jnp.float32),
            pltpu.VMEM((BLK,), jnp.int32),
        ],
    )
    def k(dst_hbm, one_hbm, z_hbm, deg_hbm, degsh, onebuf, didx):
        c = lax.axis_index("c")
        s = lax.axis_index("s")
        wid = c * NS + s
        pltpu.sync_copy(one_hbm, onebuf)
        pltpu.sync_copy(z_hbm, degsh.at[pl.ds(s * RPW, RPW)])
        plsc.subcore_barrier()
        base0 = wid * EPW

        @pl.loop(0, NBLK)
        def _(j):
            pltpu.sync_copy(dst_hbm.at[pl.ds(base0 + j * BLK, BLK)], didx)
            pltpu.sync_copy(onebuf, degsh.at[didx], add=True)

        plsc.subcore_barrier()
        pltpu.sync_copy(degsh.at[pl.ds(s * RPW, RPW)],
                        deg_hbm.at[c, pl.ds(s * RPW, RPW)])

    return k(dstp, ones16, zeros16)


def _agg_sc(srcp, dstp, ytab, disrows, zdp, z16):
    @functools.partial(
        pl.kernel,
        mesh=_MESH,
        compiler_params=_SC_PARAMS,
        out_type=(
            jax.ShapeDtypeStruct((NC, NPAD, DP), jnp.float32),
            jax.ShapeDtypeStruct((NC, NPAD, 16), jnp.float32),
        ),
        scratch_types=[
            pltpu.VMEM_SHARED((NPAD, DP), jnp.float32),
            pltpu.VMEM_SHARED((NPAD, 16), jnp.float32),
            pltpu.VMEM((BLK, DP), jnp.float32),
            pltpu.VMEM((BLK, 16), jnp.float32),
            pltpu.VMEM((BLK,), jnp.int32),
            pltpu.VMEM((BLK,), jnp.int32),
            pltpu.SemaphoreType.DMA,
        ],
    )
    def k(src_hbm, dst_hbm, y_hbm, dr_hbm, zdp_hbm, z16_hbm,
          acc_hbm, s_hbm, accsh, ssh, rows, drows, sidx, didx, sem):
        c = lax.axis_index("c")
        s = lax.axis_index("s")
        wid = c * NS + s
        pltpu.sync_copy(zdp_hbm, accsh.at[pl.ds(s * RPW, RPW)])
        pltpu.sync_copy(z16_hbm, ssh.at[pl.ds(s * RPW, RPW)])
        plsc.subcore_barrier()
        base0 = wid * EPW

        @pl.loop(0, NBLK)
        def _(j):
            b = base0 + j * BLK
            pltpu.sync_copy(src_hbm.at[pl.ds(b, BLK)], sidx)
            pltpu.sync_copy(dst_hbm.at[pl.ds(b, BLK)], didx)
            pltpu.async_copy(y_hbm.at[sidx], rows, sem).wait()
            pltpu.sync_copy(rows, accsh.at[didx], add=True)
            pltpu.async_copy(dr_hbm.at[didx], drows, sem).wait()
            pltpu.sync_copy(drows, ssh.at[sidx], add=True)

        plsc.subcore_barrier()
        pltpu.sync_copy(accsh.at[pl.ds(s * RPW, RPW)],
                        acc_hbm.at[c, pl.ds(s * RPW, RPW)])
        pltpu.sync_copy(ssh.at[pl.ds(s * RPW, RPW)],
                        s_hbm.at[c, pl.ds(s * RPW, RPW)])

    return k(srcp, dstp, ytab, disrows, zdp, z16)


def _mm_tc(xp, W1p):
    def body(x_ref, w_ref, o_ref):
        o_ref[...] = jnp.dot(x_ref[...], w_ref[...],
                             preferred_element_type=jnp.float32)

    return pl.pallas_call(
        body,
        grid=(NROWBLK,),
        in_specs=[
            pl.BlockSpec((ROWBLK, D_IN), lambda i: (i, 0)),
            pl.BlockSpec((D_IN, DP), lambda i: (0, 0)),
        ],
        out_specs=pl.BlockSpec((ROWBLK, DP), lambda i: (i, 0)),
        out_shape=jax.ShapeDtypeStruct((NPAD, DP), jnp.float32),
    )(xp, W1p)


def _prep_tc(h0, degp):
    def body(h_ref, d_ref, y_ref, dr_ref):
        deg = d_ref[0, :, 0:1] + d_ref[1, :, 0:1] + 1.0
        dis = lax.rsqrt(deg)
        y_ref[...] = h_ref[...] * dis
        dr_ref[...] = jnp.broadcast_to(dis, (ROWBLK, 16))

    return pl.pallas_call(
        body,
        grid=(NROWBLK,),
        in_specs=[
            pl.BlockSpec((ROWBLK, DP), lambda i: (i, 0)),
            pl.BlockSpec((NC, ROWBLK, 16), lambda i: (0, i, 0)),
        ],
        out_specs=[
            pl.BlockSpec((ROWBLK, DP), lambda i: (i, 0)),
            pl.BlockSpec((ROWBLK, 16), lambda i: (i, 0)),
        ],
        out_shape=[
            jax.ShapeDtypeStruct((NPAD, DP), jnp.float32),
            jax.ShapeDtypeStruct((NPAD, 16), jnp.float32),
        ],
    )(h0, degp)


def _fin_tc(accp, spart, ytab, disrows, b1p, W2p, b2p):
    def body(acc_ref, s_ref, y_ref, dr_ref, b1_ref, w2_ref, b2_ref,
             o_ref, vacc):
        i = pl.program_id(0)

        @pl.when(i == 0)
        def _():
            vacc[...] = jnp.zeros((1, DP), jnp.float32)

        dis = dr_ref[:, 0:1]
        acc = acc_ref[0] + acc_ref[1]
        h1 = dis * (acc + y_ref[...]) + b1_ref[...]
        r = jnp.maximum(h1, 0.0)
        sp = s_ref[0, :, 0:1] + s_ref[1, :, 0:1]
        sfull = dis * (sp + dis)
        rowid = i * ROWBLK + lax.broadcasted_iota(jnp.int32, (ROWBLK, 1), 0)
        sfull = jnp.where(rowid < N, sfull, 0.0)
        vacc[...] += jnp.sum(r * sfull, axis=0, keepdims=True)

        @pl.when(i == NROWBLK - 1)
        def _():
            v = vacc[...]
            pooled = jnp.dot(v, w2_ref[...],
                             preferred_element_type=jnp.float32)
            pooled = pooled * (1.0 / N) + b2_ref[...]
            laneid = lax.broadcasted_iota(jnp.int32, (1, CP), 1)
            valid = laneid < NCLS
            pm = jnp.where(valid, pooled, -1e30)
            m = jnp.max(pm, axis=1, keepdims=True)
            e = jnp.where(valid, jnp.exp(pooled - m), 0.0)
            lse = jnp.log(jnp.sum(e, axis=1, keepdims=True))
            o_ref[...] = pooled - m - lse

    return pl.pallas_call(
        body,
        grid=(NROWBLK,),
        in_specs=[
            pl.BlockSpec((NC, ROWBLK, DP), lambda i: (0, i, 0)),
            pl.BlockSpec((NC, ROWBLK, 16), lambda i: (0, i, 0)),
            pl.BlockSpec((ROWBLK, DP), lambda i: (i, 0)),
            pl.BlockSpec((ROWBLK, 16), lambda i: (i, 0)),
            pl.BlockSpec((1, DP), lambda i: (0, 0)),
            pl.BlockSpec((DP, CP), lambda i: (0, 0)),
            pl.BlockSpec((1, CP), lambda i: (0, 0)),
        ],
        out_specs=pl.BlockSpec((1, CP), lambda i: (0, 0)),
        out_shape=jax.ShapeDtypeStruct((1, CP), jnp.float32),
        scratch_shapes=[pltpu.VMEM((1, DP), jnp.float32)],
    )(accp, spart, ytab, disrows, b1p, W2p, b2p)


def kernel(x, edge_index, W1, b1, W2, b2):
    src = edge_index[0].astype(jnp.int32)
    dst = edge_index[1].astype(jnp.int32)
    pad = jnp.full((EPAD - E,), N, jnp.int32)
    srcp = jnp.concatenate([src, pad])
    dstp = jnp.concatenate([dst, pad])

    xp = jnp.pad(x.astype(jnp.float32), ((0, NPAD - N), (0, 0)))
    W1p = jnp.pad(W1.astype(jnp.float32), ((0, 0), (0, DP - D_HID)))
    b1p = jnp.pad(b1.astype(jnp.float32), (0, DP - D_HID)).reshape(1, DP)
    W2p = jnp.pad(W2.astype(jnp.float32), ((0, DP - D_HID), (0, CP - NCLS)))
    b2p = jnp.pad(b2.astype(jnp.float32), (0, CP - NCLS)).reshape(1, CP)

    ones16 = jnp.zeros((BLK, 16), jnp.float32).at[:, 0].set(1.0)
    zeros16 = jnp.zeros((RPW, 16), jnp.float32)
    zerosdp = jnp.zeros((RPW, DP), jnp.float32)

    degp = _deg_sc(dstp, ones16, zeros16)
    h0 = _mm_tc(xp, W1p)
    ytab, disrows = _prep_tc(h0, degp)
    accp, spart = _agg_sc(srcp, dstp, ytab, disrows, zerosdp, zeros16)
    out16 = _fin_tc(accp, spart, ytab, disrows, b1p, W2p, b2p)
    return out16[:, :NCLS]


# speedup vs baseline: 19.1996x; 1.4408x over previous
import functools

import jax
import jax.numpy as jnp
from jax import lax
from jax.experimental import pallas as pl
from jax.experimental.pallas import tpu as pltpu
from jax.experimental.pallas import tpu_sc as plsc

N = 10000
E = 320000
D_IN = 128
D_HID = 100
DP = 112
NCLS = 10
CP = 16

NC = 2
NS = 16
NW = NC * NS
EPW = 10240
EPAD = NW * EPW
BLK = 128
NBLK = EPW // BLK
NPAD = 10240
RPW = NPAD // NS

ROWBLK = 512
NROWBLK = NPAD // ROWBLK

_MESH = plsc.VectorSubcoreMesh(core_axis_name="c", subcore_axis_name="s")
_SC_PARAMS = pltpu.CompilerParams(use_tc_tiling_on_sc=False)


def _deg_sc(dstp, ones16, zeros16):
    @functools.partial(
        pl.kernel,
        mesh=_MESH,
        compiler_params=_SC_PARAMS,
        out_type=jax.ShapeDtypeStruct((NC, NPAD, 16), jnp.float32),
        scratch_types=[
            pltpu.VMEM_SHARED((NPAD, 16), jnp.float32),
            pltpu.VMEM((BLK, 16), jnp.float32),
            pltpu.VMEM((BLK,), jnp.int32),
            pltpu.VMEM((BLK,), jnp.int32),
            pltpu.SemaphoreType.DMA,
            pltpu.SemaphoreType.DMA,
        ],
    )
    def k(dst_hbm, one_hbm, z_hbm, deg_hbm, degsh, onebuf, didx0, didx1,
          semI0, semI1):
        c = lax.axis_index("c")
        s = lax.axis_index("s")
        wid = c * NS + s
        pltpu.sync_copy(one_hbm, onebuf)
        pltpu.sync_copy(z_hbm, degsh.at[pl.ds(s * RPW, RPW)])
        plsc.subcore_barrier()
        base0 = wid * EPW
        didx = (didx0, didx1)
        semI = (semI0, semI1)

        pltpu.async_copy(dst_hbm.at[pl.ds(base0, BLK)], didx0, semI0)

        @pl.loop(0, NBLK // 2)
        def _(jj):
            j0 = jj * 2
            for p in (0, 1):
                j = j0 + p
                q = 1 - p
                pltpu.make_async_copy(dst_hbm.at[pl.ds(0, BLK)], didx[p],
                                      semI[p]).wait()

                @pl.when(j < NBLK - 1)
                def _():
                    pltpu.async_copy(
                        dst_hbm.at[pl.ds(base0 + (j + 1) * BLK, BLK)],
                        didx[q], semI[q])

                pltpu.sync_copy(onebuf, degsh.at[didx[p]], add=True)

        plsc.subcore_barrier()
        pltpu.sync_copy(degsh.at[pl.ds(s * RPW, RPW)],
                        deg_hbm.at[c, pl.ds(s * RPW, RPW)])

    return k(dstp, ones16, zeros16)


def _agg_sc(srcp, dstp, ytab, disrows, zdp, z16):
    @functools.partial(
        pl.kernel,
        mesh=_MESH,
        compiler_params=_SC_PARAMS,
        out_type=(
            jax.ShapeDtypeStruct((NC, NPAD, DP), jnp.float32),
            jax.ShapeDtypeStruct((NC, NPAD, 16), jnp.float32),
        ),
        scratch_types=[
            pltpu.VMEM_SHARED((NPAD, DP), jnp.float32),
            pltpu.VMEM_SHARED((NPAD, 16), jnp.float32),
            pltpu.VMEM((BLK, DP), jnp.float32),
            pltpu.VMEM((BLK, DP), jnp.float32),
            pltpu.VMEM((BLK, 16), jnp.float32),
            pltpu.VMEM((BLK, 16), jnp.float32),
            pltpu.VMEM((BLK,), jnp.int32),
            pltpu.VMEM((BLK,), jnp.int32),
            pltpu.VMEM((BLK,), jnp.int32),
            pltpu.VMEM((BLK,), jnp.int32),
            pltpu.SemaphoreType.DMA,
            pltpu.SemaphoreType.DMA,
            pltpu.SemaphoreType.DMA,
            pltpu.SemaphoreType.DMA,
            pltpu.SemaphoreType.DMA,
            pltpu.SemaphoreType.DMA,
        ],
    )
    def k(src_hbm, dst_hbm, y_hbm, dr_hbm, zdp_hbm, z16_hbm,
          acc_hbm, s_hbm, accsh, ssh, rows0, rows1, drows0, drows1,
          sidx0, sidx1, didx0, didx1, semI0, semI1, semG0, semG1,
          semD0, semD1):
        c = lax.axis_index("c")
        s = lax.axis_index("s")
        wid = c * NS + s
        pltpu.sync_copy(zdp_hbm, accsh.at[pl.ds(s * RPW, RPW)])
        pltpu.sync_copy(z16_hbm, ssh.at[pl.ds(s * RPW, RPW)])
        plsc.subcore_barrier()
        base0 = wid * EPW
        rows = (rows0, rows1)
        drows = (drows0, drows1)
        sidx = (sidx0, sidx1)
        didx = (didx0, didx1)
        semI = (semI0, semI1)
        semG = (semG0, semG1)
        semD = (semD0, semD1)

        def idx_fetch(j, p):
            pltpu.async_copy(src_hbm.at[pl.ds(base0 + j * BLK, BLK)],
                             sidx[p], semI[p])
            pltpu.async_copy(dst_hbm.at[pl.ds(base0 + j * BLK, BLK)],
                             didx[p], semI[p])

        def idx_wait(p):
            pltpu.make_async_copy(src_hbm.at[pl.ds(0, BLK)], sidx[p],
                                  semI[p]).wait()
            pltpu.make_async_copy(src_hbm.at[pl.ds(0, BLK)], didx[p],
                                  semI[p]).wait()

        def gathers(p):
            pltpu.async_copy(y_hbm.at[sidx[p]], rows[p], semG[p])
            pltpu.async_copy(dr_hbm.at[didx[p]], drows[p], semD[p])

        def scatters(p):
            pltpu.make_async_copy(y_hbm.at[sidx[p]], rows[p], semG[p]).wait()
            pltpu.sync_copy(rows[p], accsh.at[didx[p]], add=True)
            pltpu.make_async_copy(dr_hbm.at[didx[p]], drows[p],
                                  semD[p]).wait()
            pltpu.sync_copy(drows[p], ssh.at[sidx[p]], add=True)

        idx_fetch(0, 0)
        idx_fetch(1, 1)
        idx_wait(0)
        gathers(0)

        @pl.loop(0, NBLK // 2)
        def _(jj):
            j0 = jj * 2
            for p in (0, 1):
                j = j0 + p
                q = 1 - p

                @pl.when(j < NBLK - 1)
                def _():
                    idx_wait(q)
                    gathers(q)

                scatters(p)

                @pl.when(j < NBLK - 2)
                def _():
                    idx_fetch(j + 2, p)

        plsc.subcore_barrier()
        pltpu.sync_copy(accsh.at[pl.ds(s * RPW, RPW)],
                        acc_hbm.at[c, pl.ds(s * RPW, RPW)])
        pltpu.sync_copy(ssh.at[pl.ds(s * RPW, RPW)],
                        s_hbm.at[c, pl.ds(s * RPW, RPW)])

    return k(srcp, dstp, ytab, disrows, zdp, z16)


def _mm_tc(xp, W1p):
    def body(x_ref, w_ref, o_ref):
        o_ref[...] = jnp.dot(x_ref[...], w_ref[...],
                             preferred_element_type=jnp.float32)

    return pl.pallas_call(
        body,
        grid=(NROWBLK,),
        in_specs=[
            pl.BlockSpec((ROWBLK, D_IN), lambda i: (i, 0)),
            pl.BlockSpec((D_IN, DP), lambda i: (0, 0)),
        ],
        out_specs=pl.BlockSpec((ROWBLK, DP), lambda i: (i, 0)),
        out_shape=jax.ShapeDtypeStruct((NPAD, DP), jnp.float32),
    )(xp, W1p)


def _prep_tc(h0, degp):
    def body(h_ref, d_ref, y_ref, dr_ref):
        deg = d_ref[0, :, 0:1] + d_ref[1, :, 0:1] + 1.0
        dis = lax.rsqrt(deg)
        y_ref[...] = h_ref[...] * dis
        dr_ref[...] = jnp.broadcast_to(dis, (ROWBLK, 16))

    return pl.pallas_call(
        body,
        grid=(NROWBLK,),
        in_specs=[
            pl.BlockSpec((ROWBLK, DP), lambda i: (i, 0)),
            pl.BlockSpec((NC, ROWBLK, 16), lambda i: (0, i, 0)),
        ],
        out_specs=[
            pl.BlockSpec((ROWBLK, DP), lambda i: (i, 0)),
            pl.BlockSpec((ROWBLK, 16), lambda i: (i, 0)),
        ],
        out_shape=[
            jax.ShapeDtypeStruct((NPAD, DP), jnp.float32),
            jax.ShapeDtypeStruct((NPAD, 16), jnp.float32),
        ],
    )(h0, degp)


def _fin_tc(accp, spart, ytab, disrows, b1p, W2p, b2p):
    def body(acc_ref, s_ref, y_ref, dr_ref, b1_ref, w2_ref, b2_ref,
             o_ref, vacc):
        i = pl.program_id(0)

        @pl.when(i == 0)
        def _():
            vacc[...] = jnp.zeros((1, DP), jnp.float32)

        dis = dr_ref[:, 0:1]
        acc = acc_ref[0] + acc_ref[1]
        h1 = dis * (acc + y_ref[...]) + b1_ref[...]
        r = jnp.maximum(h1, 0.0)
        sp = s_ref[0, :, 0:1] + s_ref[1, :, 0:1]
        sfull = dis * (sp + dis)
        rowid = i * ROWBLK + lax.broadcasted_iota(jnp.int32, (ROWBLK, 1), 0)
        sfull = jnp.where(rowid < N, sfull, 0.0)
        vacc[...] += jnp.sum(r * sfull, axis=0, keepdims=True)

        @pl.when(i == NROWBLK - 1)
        def _():
            v = vacc[...]
            pooled = jnp.dot(v, w2_ref[...],
                             preferred_element_type=jnp.float32)
            pooled = pooled * (1.0 / N) + b2_ref[...]
            laneid = lax.broadcasted_iota(jnp.int32, (1, CP), 1)
            valid = laneid < NCLS
            pm = jnp.where(valid, pooled, -1e30)
            m = jnp.max(pm, axis=1, keepdims=True)
            e = jnp.where(valid, jnp.exp(pooled - m), 0.0)
            lse = jnp.log(jnp.sum(e, axis=1, keepdims=True))
            o_ref[...] = pooled - m - lse

    return pl.pallas_call(
        body,
        grid=(NROWBLK,),
        in_specs=[
            pl.BlockSpec((NC, ROWBLK, DP), lambda i: (0, i, 0)),
            pl.BlockSpec((NC, ROWBLK, 16), lambda i: (0, i, 0)),
            pl.BlockSpec((ROWBLK, DP), lambda i: (i, 0)),
            pl.BlockSpec((ROWBLK, 16), lambda i: (i, 0)),
            pl.BlockSpec((1, DP), lambda i: (0, 0)),
            pl.BlockSpec((DP, CP), lambda i: (0, 0)),
            pl.BlockSpec((1, CP), lambda i: (0, 0)),
        ],
        out_specs=pl.BlockSpec((1, CP), lambda i: (0, 0)),
        out_shape=jax.ShapeDtypeStruct((1, CP), jnp.float32),
        scratch_shapes=[pltpu.VMEM((1, DP), jnp.float32)],
    )(accp, spart, ytab, disrows, b1p, W2p, b2p)


def kernel(x, edge_index, W1, b1, W2, b2):
    src = edge_index[0].astype(jnp.int32)
    dst = edge_index[1].astype(jnp.int32)
    pad = jnp.full((EPAD - E,), N, jnp.int32)
    srcp = jnp.concatenate([src, pad])
    dstp = jnp.concatenate([dst, pad])

    xp = jnp.pad(x.astype(jnp.float32), ((0, NPAD - N), (0, 0)))
    W1p = jnp.pad(W1.astype(jnp.float32), ((0, 0), (0, DP - D_HID)))
    b1p = jnp.pad(b1.astype(jnp.float32), (0, DP - D_HID)).reshape(1, DP)
    W2p = jnp.pad(W2.astype(jnp.float32), ((0, DP - D_HID), (0, CP - NCLS)))
    b2p = jnp.pad(b2.astype(jnp.float32), (0, CP - NCLS)).reshape(1, CP)

    ones16 = jnp.zeros((BLK, 16), jnp.float32).at[:, 0].set(1.0)
    zeros16 = jnp.zeros((RPW, 16), jnp.float32)
    zerosdp = jnp.zeros((RPW, DP), jnp.float32)

    degp = _deg_sc(dstp, ones16, zeros16)
    h0 = _mm_tc(xp, W1p)
    ytab, disrows = _prep_tc(h0, degp)
    accp, spart = _agg_sc(srcp, dstp, ytab, disrows, zerosdp, zeros16)
    out16 = _fin_tc(accp, spart, ytab, disrows, b1p, W2p, b2p)
    return out16[:, :NCLS]


# speedup vs baseline: 35.8371x; 1.8666x over previous
import functools

import jax
import jax.numpy as jnp
from jax import lax
from jax.experimental import pallas as pl
from jax.experimental.pallas import tpu as pltpu
from jax.experimental.pallas import tpu_sc as plsc

N = 10000
E = 320000
D_IN = 128
D_HID = 100
DP = 112
NCLS = 10
CP = 16

NC = 2
NS = 16
NW = NC * NS
EPW = 10240
EPAD = NW * EPW
BLK = 128
NBLK = EPW // BLK
NPAD = 10240
RPW = NPAD // NS

ROWBLK = 512
NROWBLK = NPAD // ROWBLK

_MESH = plsc.VectorSubcoreMesh(core_axis_name="c", subcore_axis_name="s")
_SC_PARAMS = pltpu.CompilerParams(use_tc_tiling_on_sc=False)


def _deg_sc(dstp, ones16, zeros16):
    @functools.partial(
        pl.kernel,
        mesh=_MESH,
        compiler_params=_SC_PARAMS,
        out_type=jax.ShapeDtypeStruct((NC, NPAD, 16), jnp.float32),
        scratch_types=[
            pltpu.VMEM_SHARED((NPAD, 16), jnp.float32),
            pltpu.VMEM((BLK, 16), jnp.float32),
            pltpu.VMEM((BLK,), jnp.int32),
            pltpu.VMEM((BLK,), jnp.int32),
            pltpu.SemaphoreType.DMA,
            pltpu.SemaphoreType.DMA,
        ],
    )
    def k(dst_hbm, one_hbm, z_hbm, deg_hbm, degsh, onebuf, didx0, didx1,
          semI0, semI1):
        c = lax.axis_index("c")
        s = lax.axis_index("s")
        wid = c * NS + s
        pltpu.sync_copy(one_hbm, onebuf)
        pltpu.sync_copy(z_hbm, degsh.at[pl.ds(s * RPW, RPW)])
        plsc.subcore_barrier()
        base0 = wid * EPW
        didx = (didx0, didx1)
        semI = (semI0, semI1)

        pltpu.async_copy(dst_hbm.at[pl.ds(base0, BLK)], didx0, semI0)

        @pl.loop(0, NBLK // 2)
        def _(jj):
            j0 = jj * 2
            for p in (0, 1):
                j = j0 + p
                q = 1 - p
                pltpu.make_async_copy(dst_hbm.at[pl.ds(0, BLK)], didx[p],
                                      semI[p]).wait()

                @pl.when(j < NBLK - 1)
                def _():
                    pltpu.async_copy(
                        dst_hbm.at[pl.ds(base0 + (j + 1) * BLK, BLK)],
                        didx[q], semI[q])

                pltpu.sync_copy(onebuf, degsh.at[didx[p]], add=True)

        plsc.subcore_barrier()
        pltpu.sync_copy(degsh.at[pl.ds(s * RPW, RPW)],
                        deg_hbm.at[c, pl.ds(s * RPW, RPW)])

    return k(dstp, ones16, zeros16)


def _agg_sc(srcp, dstp, ytab, disrows, zdp, z16):
    @functools.partial(
        pl.kernel,
        mesh=_MESH,
        compiler_params=_SC_PARAMS,
        out_type=(
            jax.ShapeDtypeStruct((NC, NPAD, DP), jnp.float32),
            jax.ShapeDtypeStruct((NC, NPAD, 16), jnp.float32),
        ),
        scratch_types=[
            pltpu.VMEM_SHARED((NPAD, DP), jnp.float32),
            pltpu.VMEM_SHARED((NPAD, 16), jnp.float32),
            pltpu.VMEM((BLK, DP), jnp.float32),
            pltpu.VMEM((BLK, DP), jnp.float32),
            pltpu.VMEM((BLK, 16), jnp.float32),
            pltpu.VMEM((BLK, 16), jnp.float32),
            pltpu.VMEM((BLK,), jnp.int32),
            pltpu.VMEM((BLK,), jnp.int32),
            pltpu.VMEM((BLK,), jnp.int32),
            pltpu.VMEM((BLK,), jnp.int32),
            pltpu.SemaphoreType.DMA,
            pltpu.SemaphoreType.DMA,
            pltpu.SemaphoreType.DMA,
            pltpu.SemaphoreType.DMA,
            pltpu.SemaphoreType.DMA,
            pltpu.SemaphoreType.DMA,
        ],
    )
    def k(src_hbm, dst_hbm, y_hbm, dr_hbm, zdp_hbm, z16_hbm,
          acc_hbm, s_hbm, accsh, ssh, rows0, rows1, drows0, drows1,
          sidx0, sidx1, didx0, didx1, semI0, semI1, semG0, semG1,
          semD0, semD1):
        c = lax.axis_index("c")
        s = lax.axis_index("s")
        wid = c * NS + s
        pltpu.sync_copy(zdp_hbm, accsh.at[pl.ds(s * RPW, RPW)])
        pltpu.sync_copy(z16_hbm, ssh.at[pl.ds(s * RPW, RPW)])
        plsc.subcore_barrier()
        base0 = wid * EPW
        rows = (rows0, rows1)
        drows = (drows0, drows1)
        sidx = (sidx0, sidx1)
        didx = (didx0, didx1)
        semI = (semI0, semI1)
        semG = (semG0, semG1)
        semD = (semD0, semD1)

        def idx_fetch(j, p):
            pltpu.async_copy(src_hbm.at[pl.ds(base0 + j * BLK, BLK)],
                             sidx[p], semI[p])
            pltpu.async_copy(dst_hbm.at[pl.ds(base0 + j * BLK, BLK)],
                             didx[p], semI[p])

        def idx_wait(p):
            pltpu.make_async_copy(src_hbm.at[pl.ds(0, BLK)], sidx[p],
                                  semI[p]).wait()
            pltpu.make_async_copy(src_hbm.at[pl.ds(0, BLK)], didx[p],
                                  semI[p]).wait()

        def gathers(p):
            pltpu.async_copy(y_hbm.at[sidx[p]], rows[p], semG[p])
            pltpu.async_copy(dr_hbm.at[didx[p]], drows[p], semD[p])

        def scatters(p):
            pltpu.make_async_copy(y_hbm.at[sidx[p]], rows[p], semG[p]).wait()
            pltpu.sync_copy(rows[p], accsh.at[didx[p]], add=True)
            pltpu.make_async_copy(dr_hbm.at[didx[p]], drows[p],
                                  semD[p]).wait()
            pltpu.sync_copy(drows[p], ssh.at[sidx[p]], add=True)

        idx_fetch(0, 0)
        idx_fetch(1, 1)
        idx_wait(0)
        gathers(0)

        @pl.loop(0, NBLK // 2)
        def _(jj):
            j0 = jj * 2
            for p in (0, 1):
                j = j0 + p
                q = 1 - p

                @pl.when(j < NBLK - 1)
                def _():
                    idx_wait(q)
                    gathers(q)

                scatters(p)

                @pl.when(j < NBLK - 2)
                def _():
                    idx_fetch(j + 2, p)

        plsc.subcore_barrier()
        pltpu.sync_copy(accsh.at[pl.ds(s * RPW, RPW)],
                        acc_hbm.at[c, pl.ds(s * RPW, RPW)])
        pltpu.sync_copy(ssh.at[pl.ds(s * RPW, RPW)],
                        s_hbm.at[c, pl.ds(s * RPW, RPW)])

    return k(srcp, dstp, ytab, disrows, zdp, z16)


def _mm_tc(xp, W1p):
    def body(x_ref, w_ref, o_ref):
        o_ref[...] = jnp.dot(x_ref[...], w_ref[...],
                             preferred_element_type=jnp.float32)

    return pl.pallas_call(
        body,
        grid=(NROWBLK,),
        in_specs=[
            pl.BlockSpec((ROWBLK, D_IN), lambda i: (i, 0)),
            pl.BlockSpec((D_IN, DP), lambda i: (0, 0)),
        ],
        out_specs=pl.BlockSpec((ROWBLK, DP), lambda i: (i, 0)),
        out_shape=jax.ShapeDtypeStruct((NPAD, DP), jnp.float32),
    )(xp, W1p)


def _prep_tc(h0, degp):
    def body(h_ref, d_ref, y_ref, dr_ref):
        deg = d_ref[0, :, 0:1] + d_ref[1, :, 0:1] + 1.0
        dis = lax.rsqrt(deg)
        y_ref[...] = h_ref[...] * dis
        dr_ref[...] = jnp.broadcast_to(dis, (ROWBLK, 16))

    return pl.pallas_call(
        body,
        grid=(NROWBLK,),
        in_specs=[
            pl.BlockSpec((ROWBLK, DP), lambda i: (i, 0)),
            pl.BlockSpec((NC, ROWBLK, 16), lambda i: (0, i, 0)),
        ],
        out_specs=[
            pl.BlockSpec((ROWBLK, DP), lambda i: (i, 0)),
            pl.BlockSpec((ROWBLK, 16), lambda i: (i, 0)),
        ],
        out_shape=[
            jax.ShapeDtypeStruct((NPAD, DP), jnp.float32),
            jax.ShapeDtypeStruct((NPAD, 16), jnp.float32),
        ],
    )(h0, degp)


def _fin_tc(accp, spart, ytab, disrows, b1p, W2p, b2p):
    def body(acc_ref, s_ref, y_ref, dr_ref, b1_ref, w2_ref, b2_ref,
             o_ref, vacc):
        i = pl.program_id(0)

        @pl.when(i == 0)
        def _():
            vacc[...] = jnp.zeros((1, DP), jnp.float32)

        dis = dr_ref[:, 0:1]
        acc = acc_ref[0] + acc_ref[1]
        h1 = dis * (acc + y_ref[...]) + b1_ref[...]
        r = jnp.maximum(h1, 0.0)
        sp = s_ref[0, :, 0:1] + s_ref[1, :, 0:1]
        sfull = dis * (sp + dis)
        rowid = i * ROWBLK + lax.broadcasted_iota(jnp.int32, (ROWBLK, 1), 0)
        sfull = jnp.where(rowid < N, sfull, 0.0)
        vacc[...] += jnp.sum(r * sfull, axis=0, keepdims=True)

        @pl.when(i == NROWBLK - 1)
        def _():
            v = vacc[...]
            pooled = jnp.dot(v, w2_ref[...],
                             preferred_element_type=jnp.float32)
            pooled = pooled * (1.0 / N) + b2_ref[...]
            laneid = lax.broadcasted_iota(jnp.int32, (1, CP), 1)
            valid = laneid < NCLS
            pm = jnp.where(valid, pooled, -1e30)
            m = jnp.max(pm, axis=1, keepdims=True)
            e = jnp.where(valid, jnp.exp(pooled - m), 0.0)
            lse = jnp.log(jnp.sum(e, axis=1, keepdims=True))
            o_ref[...] = pooled - m - lse

    return pl.pallas_call(
        body,
        grid=(NROWBLK,),
        in_specs=[
            pl.BlockSpec((NC, ROWBLK, DP), lambda i: (0, i, 0)),
            pl.BlockSpec((NC, ROWBLK, 16), lambda i: (0, i, 0)),
            pl.BlockSpec((ROWBLK, DP), lambda i: (i, 0)),
            pl.BlockSpec((ROWBLK, 16), lambda i: (i, 0)),
            pl.BlockSpec((1, DP), lambda i: (0, 0)),
            pl.BlockSpec((DP, CP), lambda i: (0, 0)),
            pl.BlockSpec((1, CP), lambda i: (0, 0)),
        ],
        out_specs=pl.BlockSpec((1, CP), lambda i: (0, 0)),
        out_shape=jax.ShapeDtypeStruct((1, CP), jnp.float32),
        scratch_shapes=[pltpu.VMEM((1, DP), jnp.float32)],
    )(accp, spart, ytab, disrows, b1p, W2p, b2p)


def kernel(x, edge_index, W1, b1, W2, b2):
    src = edge_index[0].astype(jnp.int32)
    dst = edge_index[1].astype(jnp.int32)
    pad = N + (jnp.arange(EPAD - E, dtype=jnp.int32) % (NPAD - N))
    srcp = jnp.concatenate([src, pad])
    dstp = jnp.concatenate([dst, pad])

    xp = jnp.pad(x.astype(jnp.float32), ((0, NPAD - N), (0, 0)))
    W1p = jnp.pad(W1.astype(jnp.float32), ((0, 0), (0, DP - D_HID)))
    b1p = jnp.pad(b1.astype(jnp.float32), (0, DP - D_HID)).reshape(1, DP)
    W2p = jnp.pad(W2.astype(jnp.float32), ((0, DP - D_HID), (0, CP - NCLS)))
    b2p = jnp.pad(b2.astype(jnp.float32), (0, CP - NCLS)).reshape(1, CP)

    ones16 = jnp.zeros((BLK, 16), jnp.float32).at[:, 0].set(1.0)
    zeros16 = jnp.zeros((RPW, 16), jnp.float32)
    zerosdp = jnp.zeros((RPW, DP), jnp.float32)

    degp = _deg_sc(dstp, ones16, zeros16)
    h0 = _mm_tc(xp, W1p)
    ytab, disrows = _prep_tc(h0, degp)
    accp, spart = _agg_sc(srcp, dstp, ytab, disrows, zerosdp, zeros16)
    out16 = _fin_tc(accp, spart, ytab, disrows, b1p, W2p, b2p)
    return out16[:, :NCLS]


# speedup vs baseline: 36.1036x; 1.0074x over previous
import functools

import jax
import jax.numpy as jnp
from jax import lax
from jax.experimental import pallas as pl
from jax.experimental.pallas import tpu as pltpu
from jax.experimental.pallas import tpu_sc as plsc

N = 10000
E = 320000
D_IN = 128
D_HID = 100
DP = 112
NCLS = 10
CP = 16

NC = 2
NS = 16
NW = NC * NS
BLK = 128
NBLK_ALL = E // BLK
NBLK_LO = NBLK_ALL // NW
NBLK_XTRA = NBLK_ALL - NBLK_LO * NW
NPAD = 10240
RPW = NPAD // NS


def _worker_blocks(wid):
    nblk = jnp.where(wid < NBLK_XTRA, NBLK_LO + 1, NBLK_LO)
    base = wid * NBLK_LO + jnp.minimum(wid, NBLK_XTRA)
    return base, nblk

ROWBLK = 512
NROWBLK = NPAD // ROWBLK

_MESH = plsc.VectorSubcoreMesh(core_axis_name="c", subcore_axis_name="s")
_SC_PARAMS = pltpu.CompilerParams(use_tc_tiling_on_sc=False)


def _deg_sc(dstp, ones16, zeros16):
    @functools.partial(
        pl.kernel,
        mesh=_MESH,
        compiler_params=_SC_PARAMS,
        out_type=jax.ShapeDtypeStruct((NC, NPAD, 16), jnp.float32),
        scratch_types=[
            pltpu.VMEM_SHARED((NPAD, 16), jnp.float32),
            pltpu.VMEM((BLK, 16), jnp.float32),
            pltpu.VMEM((BLK,), jnp.int32),
            pltpu.VMEM((BLK,), jnp.int32),
            pltpu.SemaphoreType.DMA,
            pltpu.SemaphoreType.DMA,
        ],
    )
    def k(dst_hbm, one_hbm, z_hbm, deg_hbm, degsh, onebuf, didx0, didx1,
          semI0, semI1):
        c = lax.axis_index("c")
        s = lax.axis_index("s")
        wid = c * NS + s
        pltpu.sync_copy(one_hbm, onebuf)
        pltpu.sync_copy(z_hbm, degsh.at[pl.ds(s * RPW, RPW)])
        plsc.subcore_barrier()
        base0, nblk = _worker_blocks(wid)
        didx = (didx0, didx1)
        semI = (semI0, semI1)

        pltpu.async_copy(dst_hbm.at[pl.ds(base0 * BLK, BLK)], didx0, semI0)

        @pl.loop(0, (NBLK_LO + 2) // 2)
        def _(jj):
            j0 = jj * 2
            for p in (0, 1):
                j = j0 + p
                q = 1 - p

                @pl.when(j < nblk)
                def _():
                    pltpu.make_async_copy(dst_hbm.at[pl.ds(0, BLK)], didx[p],
                                          semI[p]).wait()

                    @pl.when(j < nblk - 1)
                    def _():
                        pltpu.async_copy(
                            dst_hbm.at[pl.ds((base0 + j + 1) * BLK, BLK)],
                            didx[q], semI[q])

                    pltpu.sync_copy(onebuf, degsh.at[didx[p]], add=True)

        plsc.subcore_barrier()
        pltpu.sync_copy(degsh.at[pl.ds(s * RPW, RPW)],
                        deg_hbm.at[c, pl.ds(s * RPW, RPW)])

    return k(dstp, ones16, zeros16)


def _agg_sc(srcp, dstp, ytab, disrows, zdp, z16):
    @functools.partial(
        pl.kernel,
        mesh=_MESH,
        compiler_params=_SC_PARAMS,
        out_type=(
            jax.ShapeDtypeStruct((NC, NPAD, DP), jnp.float32),
            jax.ShapeDtypeStruct((NC, NPAD, 16), jnp.float32),
        ),
        scratch_types=[
            pltpu.VMEM_SHARED((NPAD, DP), jnp.float32),
            pltpu.VMEM_SHARED((NPAD, 16), jnp.float32),
            pltpu.VMEM((BLK, DP), jnp.float32),
            pltpu.VMEM((BLK, DP), jnp.float32),
            pltpu.VMEM((BLK, 16), jnp.float32),
            pltpu.VMEM((BLK, 16), jnp.float32),
            pltpu.VMEM((BLK,), jnp.int32),
            pltpu.VMEM((BLK,), jnp.int32),
            pltpu.VMEM((BLK,), jnp.int32),
            pltpu.VMEM((BLK,), jnp.int32),
            pltpu.SemaphoreType.DMA,
            pltpu.SemaphoreType.DMA,
            pltpu.SemaphoreType.DMA,
            pltpu.SemaphoreType.DMA,
            pltpu.SemaphoreType.DMA,
            pltpu.SemaphoreType.DMA,
        ],
    )
    def k(src_hbm, dst_hbm, y_hbm, dr_hbm, zdp_hbm, z16_hbm,
          acc_hbm, s_hbm, accsh, ssh, rows0, rows1, drows0, drows1,
          sidx0, sidx1, didx0, didx1, semI0, semI1, semG0, semG1,
          semD0, semD1):
        c = lax.axis_index("c")
        s = lax.axis_index("s")
        wid = c * NS + s
        pltpu.sync_copy(zdp_hbm, accsh.at[pl.ds(s * RPW, RPW)])
        pltpu.sync_copy(z16_hbm, ssh.at[pl.ds(s * RPW, RPW)])
        plsc.subcore_barrier()
        base0, nblk = _worker_blocks(wid)
        rows = (rows0, rows1)
        drows = (drows0, drows1)
        sidx = (sidx0, sidx1)
        didx = (didx0, didx1)
        semI = (semI0, semI1)
        semG = (semG0, semG1)
        semD = (semD0, semD1)

        def idx_fetch(j, p):
            pltpu.async_copy(src_hbm.at[pl.ds((base0 + j) * BLK, BLK)],
                             sidx[p], semI[p])
            pltpu.async_copy(dst_hbm.at[pl.ds((base0 + j) * BLK, BLK)],
                             didx[p], semI[p])

        def idx_wait(p):
            pltpu.make_async_copy(src_hbm.at[pl.ds(0, BLK)], sidx[p],
                                  semI[p]).wait()
            pltpu.make_async_copy(src_hbm.at[pl.ds(0, BLK)], didx[p],
                                  semI[p]).wait()

        def gathers(p):
            pltpu.async_copy(y_hbm.at[sidx[p]], rows[p], semG[p])
            pltpu.async_copy(dr_hbm.at[didx[p]], drows[p], semD[p])

        def scatters(p):
            pltpu.make_async_copy(y_hbm.at[sidx[p]], rows[p], semG[p]).wait()
            pltpu.sync_copy(rows[p], accsh.at[didx[p]], add=True)
            pltpu.make_async_copy(dr_hbm.at[didx[p]], drows[p],
                                  semD[p]).wait()
            pltpu.sync_copy(drows[p], ssh.at[sidx[p]], add=True)

        idx_fetch(0, 0)
        idx_fetch(1, 1)
        idx_wait(0)
        gathers(0)

        @pl.loop(0, (NBLK_LO + 2) // 2)
        def _(jj):
            j0 = jj * 2
            for p in (0, 1):
                j = j0 + p
                q = 1 - p

                @pl.when(j < nblk)
                def _():
                    @pl.when(j < nblk - 1)
                    def _():
                        idx_wait(q)
                        gathers(q)

                    scatters(p)

                    @pl.when(j < nblk - 2)
                    def _():
                        idx_fetch(j + 2, p)

        plsc.subcore_barrier()
        pltpu.sync_copy(accsh.at[pl.ds(s * RPW, RPW)],
                        acc_hbm.at[c, pl.ds(s * RPW, RPW)])
        pltpu.sync_copy(ssh.at[pl.ds(s * RPW, RPW)],
                        s_hbm.at[c, pl.ds(s * RPW, RPW)])

    return k(srcp, dstp, ytab, disrows, zdp, z16)


def _mm_tc(xp, W1p):
    def body(x_ref, w_ref, o_ref):
        o_ref[...] = jnp.dot(x_ref[...], w_ref[...],
                             preferred_element_type=jnp.float32)

    return pl.pallas_call(
        body,
        grid=(NROWBLK,),
        in_specs=[
            pl.BlockSpec((ROWBLK, D_IN), lambda i: (i, 0)),
            pl.BlockSpec((D_IN, DP), lambda i: (0, 0)),
        ],
        out_specs=pl.BlockSpec((ROWBLK, DP), lambda i: (i, 0)),
        out_shape=jax.ShapeDtypeStruct((NPAD, DP), jnp.float32),
    )(xp, W1p)


def _prep_tc(h0, degp):
    def body(h_ref, d_ref, y_ref, dr_ref):
        deg = d_ref[0, :, 0:1] + d_ref[1, :, 0:1] + 1.0
        dis = lax.rsqrt(deg)
        y_ref[...] = h_ref[...] * dis
        dr_ref[...] = jnp.broadcast_to(dis, (ROWBLK, 16))

    return pl.pallas_call(
        body,
        grid=(NROWBLK,),
        in_specs=[
            pl.BlockSpec((ROWBLK, DP), lambda i: (i, 0)),
            pl.BlockSpec((NC, ROWBLK, 16), lambda i: (0, i, 0)),
        ],
        out_specs=[
            pl.BlockSpec((ROWBLK, DP), lambda i: (i, 0)),
            pl.BlockSpec((ROWBLK, 16), lambda i: (i, 0)),
        ],
        out_shape=[
            jax.ShapeDtypeStruct((NPAD, DP), jnp.float32),
            jax.ShapeDtypeStruct((NPAD, 16), jnp.float32),
        ],
    )(h0, degp)


def _fin_tc(accp, spart, ytab, disrows, b1p, W2p, b2p):
    def body(acc_ref, s_ref, y_ref, dr_ref, b1_ref, w2_ref, b2_ref,
             o_ref, vacc):
        i = pl.program_id(0)

        @pl.when(i == 0)
        def _():
            vacc[...] = jnp.zeros((1, DP), jnp.float32)

        dis = dr_ref[:, 0:1]
        acc = acc_ref[0] + acc_ref[1]
        h1 = dis * (acc + y_ref[...]) + b1_ref[...]
        r = jnp.maximum(h1, 0.0)
        sp = s_ref[0, :, 0:1] + s_ref[1, :, 0:1]
        sfull = dis * (sp + dis)
        rowid = i * ROWBLK + lax.broadcasted_iota(jnp.int32, (ROWBLK, 1), 0)
        sfull = jnp.where(rowid < N, sfull, 0.0)
        vacc[...] += jnp.sum(r * sfull, axis=0, keepdims=True)

        @pl.when(i == NROWBLK - 1)
        def _():
            v = vacc[...]
            pooled = jnp.dot(v, w2_ref[...],
                             preferred_element_type=jnp.float32)
            pooled = pooled * (1.0 / N) + b2_ref[...]
            laneid = lax.broadcasted_iota(jnp.int32, (1, CP), 1)
            valid = laneid < NCLS
            pm = jnp.where(valid, pooled, -1e30)
            m = jnp.max(pm, axis=1, keepdims=True)
            e = jnp.where(valid, jnp.exp(pooled - m), 0.0)
            lse = jnp.log(jnp.sum(e, axis=1, keepdims=True))
            o_ref[...] = pooled - m - lse

    return pl.pallas_call(
        body,
        grid=(NROWBLK,),
        in_specs=[
            pl.BlockSpec((NC, ROWBLK, DP), lambda i: (0, i, 0)),
            pl.BlockSpec((NC, ROWBLK, 16), lambda i: (0, i, 0)),
            pl.BlockSpec((ROWBLK, DP), lambda i: (i, 0)),
            pl.BlockSpec((ROWBLK, 16), lambda i: (i, 0)),
            pl.BlockSpec((1, DP), lambda i: (0, 0)),
            pl.BlockSpec((DP, CP), lambda i: (0, 0)),
            pl.BlockSpec((1, CP), lambda i: (0, 0)),
        ],
        out_specs=pl.BlockSpec((1, CP), lambda i: (0, 0)),
        out_shape=jax.ShapeDtypeStruct((1, CP), jnp.float32),
        scratch_shapes=[pltpu.VMEM((1, DP), jnp.float32)],
    )(accp, spart, ytab, disrows, b1p, W2p, b2p)


def kernel(x, edge_index, W1, b1, W2, b2):
    src = edge_index[0].astype(jnp.int32)
    dst = edge_index[1].astype(jnp.int32)

    xp = jnp.pad(x.astype(jnp.float32), ((0, NPAD - N), (0, 0)))
    W1p = jnp.pad(W1.astype(jnp.float32), ((0, 0), (0, DP - D_HID)))
    b1p = jnp.pad(b1.astype(jnp.float32), (0, DP - D_HID)).reshape(1, DP)
    W2p = jnp.pad(W2.astype(jnp.float32), ((0, DP - D_HID), (0, CP - NCLS)))
    b2p = jnp.pad(b2.astype(jnp.float32), (0, CP - NCLS)).reshape(1, CP)

    ones16 = jnp.zeros((BLK, 16), jnp.float32).at[:, 0].set(1.0)
    zeros16 = jnp.zeros((RPW, 16), jnp.float32)
    zerosdp = jnp.zeros((RPW, DP), jnp.float32)

    degp = _deg_sc(dst, ones16, zeros16)
    h0 = _mm_tc(xp, W1p)
    ytab, disrows = _prep_tc(h0, degp)
    accp, spart = _agg_sc(src, dst, ytab, disrows, zerosdp, zeros16)
    out16 = _fin_tc(accp, spart, ytab, disrows, b1p, W2p, b2p)
    return out16[:, :NCLS]


# speedup vs baseline: 39.2837x; 1.0881x over previous
import functools

import jax
import jax.numpy as jnp
from jax import lax
from jax.experimental import pallas as pl
from jax.experimental.pallas import tpu as pltpu
from jax.experimental.pallas import tpu_sc as plsc

N = 10000
E = 320000
D_IN = 128
D_HID = 100
DP = 112
NCLS = 10
CP = 16

NC = 2
NS = 16
NW = NC * NS
BLK = 128
NBLK_ALL = E // BLK
NBLK_LO = NBLK_ALL // NW
NBLK_XTRA = NBLK_ALL - NBLK_LO * NW
NPAD = 10240
RPW = NPAD // NS


def _worker_blocks(wid):
    nblk = jnp.where(wid < NBLK_XTRA, NBLK_LO + 1, NBLK_LO)
    base = wid * NBLK_LO + jnp.minimum(wid, NBLK_XTRA)
    return base, nblk

ROWBLK = 2048
NROWBLK = NPAD // ROWBLK

_MESH = plsc.VectorSubcoreMesh(core_axis_name="c", subcore_axis_name="s")
_SC_PARAMS = pltpu.CompilerParams(use_tc_tiling_on_sc=False)


def _deg_sc(ei, ones16, zeros16):
    @functools.partial(
        pl.kernel,
        mesh=_MESH,
        compiler_params=_SC_PARAMS,
        out_type=jax.ShapeDtypeStruct((NC, NPAD, 16), jnp.float32),
        scratch_types=[
            pltpu.VMEM_SHARED((NPAD, 16), jnp.float32),
            pltpu.VMEM((BLK, 16), jnp.float32),
            pltpu.VMEM((BLK,), jnp.int32),
            pltpu.VMEM((BLK,), jnp.int32),
            pltpu.SemaphoreType.DMA,
            pltpu.SemaphoreType.DMA,
        ],
    )
    def k(ei_hbm, one_hbm, z_hbm, deg_hbm, degsh, onebuf, didx0, didx1,
          semI0, semI1):
        dst_hbm = ei_hbm.at[1]
        c = lax.axis_index("c")
        s = lax.axis_index("s")
        wid = c * NS + s
        pltpu.sync_copy(one_hbm, onebuf)
        pltpu.sync_copy(z_hbm, degsh.at[pl.ds(s * RPW, RPW)])
        plsc.subcore_barrier()
        base0, nblk = _worker_blocks(wid)
        didx = (didx0, didx1)
        semI = (semI0, semI1)

        pltpu.async_copy(dst_hbm.at[pl.ds(base0 * BLK, BLK)], didx0, semI0)

        @pl.loop(0, (NBLK_LO + 2) // 2)
        def _(jj):
            j0 = jj * 2
            for p in (0, 1):
                j = j0 + p
                q = 1 - p

                @pl.when(j < nblk)
                def _():
                    pltpu.make_async_copy(dst_hbm.at[pl.ds(0, BLK)], didx[p],
                                          semI[p]).wait()

                    @pl.when(j < nblk - 1)
                    def _():
                        pltpu.async_copy(
                            dst_hbm.at[pl.ds((base0 + j + 1) * BLK, BLK)],
                            didx[q], semI[q])

                    pltpu.sync_copy(onebuf, degsh.at[didx[p]], add=True)

        plsc.subcore_barrier()
        pltpu.sync_copy(degsh.at[pl.ds(s * RPW, RPW)],
                        deg_hbm.at[c, pl.ds(s * RPW, RPW)])

    return k(ei, ones16, zeros16)


def _agg_sc(ei, ytab, disrows, zdp, z16):
    @functools.partial(
        pl.kernel,
        mesh=_MESH,
        compiler_params=_SC_PARAMS,
        out_type=(
            jax.ShapeDtypeStruct((NC, NPAD, DP), jnp.float32),
            jax.ShapeDtypeStruct((NC, NPAD, 16), jnp.float32),
        ),
        scratch_types=[
            pltpu.VMEM_SHARED((NPAD, DP), jnp.float32),
            pltpu.VMEM_SHARED((NPAD, 16), jnp.float32),
            pltpu.VMEM((BLK, DP), jnp.float32),
            pltpu.VMEM((BLK, DP), jnp.float32),
            pltpu.VMEM((BLK, 16), jnp.float32),
            pltpu.VMEM((BLK, 16), jnp.float32),
            pltpu.VMEM((BLK,), jnp.int32),
            pltpu.VMEM((BLK,), jnp.int32),
            pltpu.VMEM((BLK,), jnp.int32),
            pltpu.VMEM((BLK,), jnp.int32),
            pltpu.SemaphoreType.DMA,
            pltpu.SemaphoreType.DMA,
            pltpu.SemaphoreType.DMA,
            pltpu.SemaphoreType.DMA,
            pltpu.SemaphoreType.DMA,
            pltpu.SemaphoreType.DMA,
        ],
    )
    def k(ei_hbm, y_hbm, dr_hbm, zdp_hbm, z16_hbm,
          acc_hbm, s_hbm, accsh, ssh, rows0, rows1, drows0, drows1,
          sidx0, sidx1, didx0, didx1, semI0, semI1, semG0, semG1,
          semD0, semD1):
        src_hbm = ei_hbm.at[0]
        dst_hbm = ei_hbm.at[1]
        c = lax.axis_index("c")
        s = lax.axis_index("s")
        wid = c * NS + s
        pltpu.sync_copy(zdp_hbm, accsh.at[pl.ds(s * RPW, RPW)])
        pltpu.sync_copy(z16_hbm, ssh.at[pl.ds(s * RPW, RPW)])
        plsc.subcore_barrier()
        base0, nblk = _worker_blocks(wid)
        rows = (rows0, rows1)
        drows = (drows0, drows1)
        sidx = (sidx0, sidx1)
        didx = (didx0, didx1)
        semI = (semI0, semI1)
        semG = (semG0, semG1)
        semD = (semD0, semD1)

        def idx_fetch(j, p):
            pltpu.async_copy(src_hbm.at[pl.ds((base0 + j) * BLK, BLK)],
                             sidx[p], semI[p])
            pltpu.async_copy(dst_hbm.at[pl.ds((base0 + j) * BLK, BLK)],
                             didx[p], semI[p])

        def idx_wait(p):
            pltpu.make_async_copy(src_hbm.at[pl.ds(0, BLK)], sidx[p],
                                  semI[p]).wait()
            pltpu.make_async_copy(src_hbm.at[pl.ds(0, BLK)], didx[p],
                                  semI[p]).wait()

        def gathers(p):
            pltpu.async_copy(y_hbm.at[sidx[p]], rows[p], semG[p])
            pltpu.async_copy(dr_hbm.at[didx[p]], drows[p], semD[p])

        def scatters(p):
            pltpu.make_async_copy(y_hbm.at[sidx[p]], rows[p], semG[p]).wait()
            pltpu.sync_copy(rows[p], accsh.at[didx[p]], add=True)
            pltpu.make_async_copy(dr_hbm.at[didx[p]], drows[p],
                                  semD[p]).wait()
            pltpu.sync_copy(drows[p], ssh.at[sidx[p]], add=True)

        idx_fetch(0, 0)
        idx_fetch(1, 1)
        idx_wait(0)
        gathers(0)

        @pl.loop(0, (NBLK_LO + 2) // 2)
        def _(jj):
            j0 = jj * 2
            for p in (0, 1):
                j = j0 + p
                q = 1 - p

                @pl.when(j < nblk)
                def _():
                    @pl.when(j < nblk - 1)
                    def _():
                        idx_wait(q)
                        gathers(q)

                    scatters(p)

                    @pl.when(j < nblk - 2)
                    def _():
                        idx_fetch(j + 2, p)

        plsc.subcore_barrier()
        pltpu.sync_copy(accsh.at[pl.ds(s * RPW, RPW)],
                        acc_hbm.at[c, pl.ds(s * RPW, RPW)])
        pltpu.sync_copy(ssh.at[pl.ds(s * RPW, RPW)],
                        s_hbm.at[c, pl.ds(s * RPW, RPW)])

    return k(ei, ytab, disrows, zdp, z16)


def _mm_tc(xp, W1p):
    def body(x_ref, w_ref, o_ref):
        o_ref[...] = jnp.dot(x_ref[...], w_ref[...],
                             preferred_element_type=jnp.float32)

    return pl.pallas_call(
        body,
        grid=(NROWBLK,),
        in_specs=[
            pl.BlockSpec((ROWBLK, D_IN), lambda i: (i, 0)),
            pl.BlockSpec((D_IN, DP), lambda i: (0, 0)),
        ],
        out_specs=pl.BlockSpec((ROWBLK, DP), lambda i: (i, 0)),
        out_shape=jax.ShapeDtypeStruct((NPAD, DP), jnp.float32),
    )(xp, W1p)


def _prep_tc(h0, degp):
    def body(h_ref, d_ref, y_ref, dr_ref):
        deg = d_ref[0, :, 0:1] + d_ref[1, :, 0:1] + 1.0
        dis = lax.rsqrt(deg)
        y_ref[...] = h_ref[...] * dis
        dr_ref[...] = jnp.broadcast_to(dis, (ROWBLK, 16))

    return pl.pallas_call(
        body,
        grid=(NROWBLK,),
        in_specs=[
            pl.BlockSpec((ROWBLK, DP), lambda i: (i, 0)),
            pl.BlockSpec((NC, ROWBLK, 16), lambda i: (0, i, 0)),
        ],
        out_specs=[
            pl.BlockSpec((ROWBLK, DP), lambda i: (i, 0)),
            pl.BlockSpec((ROWBLK, 16), lambda i: (i, 0)),
        ],
        out_shape=[
            jax.ShapeDtypeStruct((NPAD, DP), jnp.float32),
            jax.ShapeDtypeStruct((NPAD, 16), jnp.float32),
        ],
    )(h0, degp)


def _fin_tc(accp, spart, ytab, disrows, b1p, W2p, b2p):
    def body(acc_ref, s_ref, y_ref, dr_ref, b1_ref, w2_ref, b2_ref,
             o_ref, vacc):
        i = pl.program_id(0)

        @pl.when(i == 0)
        def _():
            vacc[...] = jnp.zeros((1, DP), jnp.float32)

        dis = dr_ref[:, 0:1]
        acc = acc_ref[0] + acc_ref[1]
        h1 = dis * (acc + y_ref[...]) + b1_ref[...]
        r = jnp.maximum(h1, 0.0)
        sp = s_ref[0, :, 0:1] + s_ref[1, :, 0:1]
        sfull = dis * (sp + dis)
        rowid = i * ROWBLK + lax.broadcasted_iota(jnp.int32, (ROWBLK, 1), 0)
        sfull = jnp.where(rowid < N, sfull, 0.0)
        vacc[...] += jnp.sum(r * sfull, axis=0, keepdims=True)

        @pl.when(i == NROWBLK - 1)
        def _():
            v = vacc[...]
            pooled = jnp.dot(v, w2_ref[...],
                             preferred_element_type=jnp.float32)
            pooled = pooled * (1.0 / N) + b2_ref[...]
            laneid = lax.broadcasted_iota(jnp.int32, (1, CP), 1)
            valid = laneid < NCLS
            pm = jnp.where(valid, pooled, -1e30)
            m = jnp.max(pm, axis=1, keepdims=True)
            e = jnp.where(valid, jnp.exp(pooled - m), 0.0)
            lse = jnp.log(jnp.sum(e, axis=1, keepdims=True))
            o_ref[...] = pooled - m - lse

    return pl.pallas_call(
        body,
        grid=(NROWBLK,),
        in_specs=[
            pl.BlockSpec((NC, ROWBLK, DP), lambda i: (0, i, 0)),
            pl.BlockSpec((NC, ROWBLK, 16), lambda i: (0, i, 0)),
            pl.BlockSpec((ROWBLK, DP), lambda i: (i, 0)),
            pl.BlockSpec((ROWBLK, 16), lambda i: (i, 0)),
            pl.BlockSpec((1, DP), lambda i: (0, 0)),
            pl.BlockSpec((DP, CP), lambda i: (0, 0)),
            pl.BlockSpec((1, CP), lambda i: (0, 0)),
        ],
        out_specs=pl.BlockSpec((1, CP), lambda i: (0, 0)),
        out_shape=jax.ShapeDtypeStruct((1, CP), jnp.float32),
        scratch_shapes=[pltpu.VMEM((1, DP), jnp.float32)],
    )(accp, spart, ytab, disrows, b1p, W2p, b2p)


def kernel(x, edge_index, W1, b1, W2, b2):
    ei = edge_index.astype(jnp.int32)

    xp = jnp.pad(x.astype(jnp.float32), ((0, NPAD - N), (0, 0)))
    W1p = jnp.pad(W1.astype(jnp.float32), ((0, 0), (0, DP - D_HID)))
    b1p = jnp.pad(b1.astype(jnp.float32), (0, DP - D_HID)).reshape(1, DP)
    W2p = jnp.pad(W2.astype(jnp.float32), ((0, DP - D_HID), (0, CP - NCLS)))
    b2p = jnp.pad(b2.astype(jnp.float32), (0, CP - NCLS)).reshape(1, CP)

    ones16 = jnp.zeros((BLK, 16), jnp.float32).at[:, 0].set(1.0)
    zeros16 = jnp.zeros((RPW, 16), jnp.float32)
    zerosdp = jnp.zeros((RPW, DP), jnp.float32)

    degp = _deg_sc(ei, ones16, zeros16)
    h0 = _mm_tc(xp, W1p)
    ytab, disrows = _prep_tc(h0, degp)
    accp, spart = _agg_sc(ei, ytab, disrows, zerosdp, zeros16)
    out16 = _fin_tc(accp, spart, ytab, disrows, b1p, W2p, b2p)
    return out16[:, :NCLS]


# speedup vs baseline: 40.2827x; 1.0254x over previous
import functools

import jax
import jax.numpy as jnp
from jax import lax
from jax.experimental import pallas as pl
from jax.experimental.pallas import tpu as pltpu
from jax.experimental.pallas import tpu_sc as plsc

N = 10000
E = 320000
D_IN = 128
D_HID = 100
DP = 128
NCLS = 10
CP = 16

NC = 2
NS = 16
NW = NC * NS
BLK = 128
NBLK_ALL = E // BLK
NBLK_LO = NBLK_ALL // NW
NBLK_XTRA = NBLK_ALL - NBLK_LO * NW
NPAD = 10240
RPW = NPAD // NS


def _worker_blocks(wid):
    nblk = jnp.where(wid < NBLK_XTRA, NBLK_LO + 1, NBLK_LO)
    base = wid * NBLK_LO + jnp.minimum(wid, NBLK_XTRA)
    return base, nblk

ROWBLK = 2048
NROWBLK = NPAD // ROWBLK

_MESH = plsc.VectorSubcoreMesh(core_axis_name="c", subcore_axis_name="s")
_SC_PARAMS = pltpu.CompilerParams(use_tc_tiling_on_sc=False)


def _deg_sc(ei, ones16, zeros16):
    @functools.partial(
        pl.kernel,
        mesh=_MESH,
        compiler_params=_SC_PARAMS,
        out_type=jax.ShapeDtypeStruct((NC, NPAD, 16), jnp.float32),
        scratch_types=[
            pltpu.VMEM_SHARED((NPAD, 16), jnp.float32),
            pltpu.VMEM((BLK, 16), jnp.float32),
            pltpu.VMEM((BLK,), jnp.int32),
            pltpu.VMEM((BLK,), jnp.int32),
            pltpu.SemaphoreType.DMA,
            pltpu.SemaphoreType.DMA,
        ],
    )
    def k(ei_hbm, one_hbm, z_hbm, deg_hbm, degsh, onebuf, didx0, didx1,
          semI0, semI1):
        dst_hbm = ei_hbm.at[1]
        c = lax.axis_index("c")
        s = lax.axis_index("s")
        wid = c * NS + s
        pltpu.sync_copy(one_hbm, onebuf)
        pltpu.sync_copy(z_hbm, degsh.at[pl.ds(s * RPW, RPW)])
        plsc.subcore_barrier()
        base0, nblk = _worker_blocks(wid)
        didx = (didx0, didx1)
        semI = (semI0, semI1)

        pltpu.async_copy(dst_hbm.at[pl.ds(base0 * BLK, BLK)], didx0, semI0)

        @pl.loop(0, (NBLK_LO + 2) // 2)
        def _(jj):
            j0 = jj * 2
            for p in (0, 1):
                j = j0 + p
                q = 1 - p

                @pl.when(j < nblk)
                def _():
                    pltpu.make_async_copy(dst_hbm.at[pl.ds(0, BLK)], didx[p],
                                          semI[p]).wait()

                    @pl.when(j < nblk - 1)
                    def _():
                        pltpu.async_copy(
                            dst_hbm.at[pl.ds((base0 + j + 1) * BLK, BLK)],
                            didx[q], semI[q])

                    pltpu.sync_copy(onebuf, degsh.at[didx[p]], add=True)

        plsc.subcore_barrier()
        pltpu.sync_copy(degsh.at[pl.ds(s * RPW, RPW)],
                        deg_hbm.at[c, pl.ds(s * RPW, RPW)])

    return k(ei, ones16, zeros16)


def _agg_sc(ei, ytab, disrows, zdp, z16):
    @functools.partial(
        pl.kernel,
        mesh=_MESH,
        compiler_params=_SC_PARAMS,
        out_type=(
            jax.ShapeDtypeStruct((NC, NPAD, DP), jnp.float32),
            jax.ShapeDtypeStruct((NC, NPAD, 16), jnp.float32),
        ),
        scratch_types=[
            pltpu.VMEM_SHARED((NPAD, DP), jnp.float32),
            pltpu.VMEM_SHARED((NPAD, 16), jnp.float32),
            pltpu.VMEM((BLK, DP), jnp.float32),
            pltpu.VMEM((BLK, DP), jnp.float32),
            pltpu.VMEM((BLK, 16), jnp.float32),
            pltpu.VMEM((BLK, 16), jnp.float32),
            pltpu.VMEM((BLK,), jnp.int32),
            pltpu.VMEM((BLK,), jnp.int32),
            pltpu.VMEM((BLK,), jnp.int32),
            pltpu.VMEM((BLK,), jnp.int32),
            pltpu.SemaphoreType.DMA,
            pltpu.SemaphoreType.DMA,
            pltpu.SemaphoreType.DMA,
            pltpu.SemaphoreType.DMA,
            pltpu.SemaphoreType.DMA,
            pltpu.SemaphoreType.DMA,
        ],
    )
    def k(ei_hbm, y_hbm, dr_hbm, zdp_hbm, z16_hbm,
          acc_hbm, s_hbm, accsh, ssh, rows0, rows1, drows0, drows1,
          sidx0, sidx1, didx0, didx1, semI0, semI1, semG0, semG1,
          semD0, semD1):
        src_hbm = ei_hbm.at[0]
        dst_hbm = ei_hbm.at[1]
        c = lax.axis_index("c")
        s = lax.axis_index("s")
        wid = c * NS + s
        pltpu.sync_copy(zdp_hbm, accsh.at[pl.ds(s * RPW, RPW)])
        pltpu.sync_copy(z16_hbm, ssh.at[pl.ds(s * RPW, RPW)])
        plsc.subcore_barrier()
        base0, nblk = _worker_blocks(wid)
        rows = (rows0, rows1)
        drows = (drows0, drows1)
        sidx = (sidx0, sidx1)
        didx = (didx0, didx1)
        semI = (semI0, semI1)
        semG = (semG0, semG1)
        semD = (semD0, semD1)

        def idx_fetch(j, p):
            pltpu.async_copy(src_hbm.at[pl.ds((base0 + j) * BLK, BLK)],
                             sidx[p], semI[p])
            pltpu.async_copy(dst_hbm.at[pl.ds((base0 + j) * BLK, BLK)],
                             didx[p], semI[p])

        def idx_wait(p):
            pltpu.make_async_copy(src_hbm.at[pl.ds(0, BLK)], sidx[p],
                                  semI[p]).wait()
            pltpu.make_async_copy(src_hbm.at[pl.ds(0, BLK)], didx[p],
                                  semI[p]).wait()

        def gathers(p):
            pltpu.async_copy(y_hbm.at[sidx[p]], rows[p], semG[p])
            pltpu.async_copy(dr_hbm.at[didx[p]], drows[p], semD[p])

        def scatters(p):
            pltpu.make_async_copy(y_hbm.at[sidx[p]], rows[p], semG[p]).wait()
            pltpu.sync_copy(rows[p], accsh.at[didx[p]], add=True)
            pltpu.make_async_copy(dr_hbm.at[didx[p]], drows[p],
                                  semD[p]).wait()
            pltpu.sync_copy(drows[p], ssh.at[sidx[p]], add=True)

        idx_fetch(0, 0)
        idx_fetch(1, 1)
        idx_wait(0)
        gathers(0)

        @pl.loop(0, (NBLK_LO + 2) // 2)
        def _(jj):
            j0 = jj * 2
            for p in (0, 1):
                j = j0 + p
                q = 1 - p

                @pl.when(j < nblk)
                def _():
                    @pl.when(j < nblk - 1)
                    def _():
                        idx_wait(q)
                        gathers(q)

                    scatters(p)

                    @pl.when(j < nblk - 2)
                    def _():
                        idx_fetch(j + 2, p)

        plsc.subcore_barrier()
        pltpu.sync_copy(accsh.at[pl.ds(s * RPW, RPW)],
                        acc_hbm.at[c, pl.ds(s * RPW, RPW)])
        pltpu.sync_copy(ssh.at[pl.ds(s * RPW, RPW)],
                        s_hbm.at[c, pl.ds(s * RPW, RPW)])

    return k(ei, ytab, disrows, zdp, z16)


def _mm_tc(xp, W1p):
    def body(x_ref, w_ref, o_ref):
        o_ref[...] = jnp.dot(x_ref[...], w_ref[...],
                             preferred_element_type=jnp.float32)

    return pl.pallas_call(
        body,
        grid=(NROWBLK,),
        in_specs=[
            pl.BlockSpec((ROWBLK, D_IN), lambda i: (i, 0)),
            pl.BlockSpec((D_IN, DP), lambda i: (0, 0)),
        ],
        out_specs=pl.BlockSpec((ROWBLK, DP), lambda i: (i, 0)),
        out_shape=jax.ShapeDtypeStruct((NPAD, DP), jnp.float32),
    )(xp, W1p)


def _prep_tc(h0, degp):
    def body(h_ref, d_ref, y_ref, dr_ref):
        deg = d_ref[0, :, 0:1] + d_ref[1, :, 0:1] + 1.0
        dis = lax.rsqrt(deg)
        y_ref[...] = h_ref[...] * dis
        dr_ref[...] = jnp.broadcast_to(dis, (ROWBLK, 16))

    return pl.pallas_call(
        body,
        grid=(NROWBLK,),
        in_specs=[
            pl.BlockSpec((ROWBLK, DP), lambda i: (i, 0)),
            pl.BlockSpec((NC, ROWBLK, 16), lambda i: (0, i, 0)),
        ],
        out_specs=[
            pl.BlockSpec((ROWBLK, DP), lambda i: (i, 0)),
            pl.BlockSpec((ROWBLK, 16), lambda i: (i, 0)),
        ],
        out_shape=[
            jax.ShapeDtypeStruct((NPAD, DP), jnp.float32),
            jax.ShapeDtypeStruct((NPAD, 16), jnp.float32),
        ],
    )(h0, degp)


def _fin_tc(accp, spart, ytab, disrows, b1p, W2p, b2p):
    def body(acc_ref, s_ref, y_ref, dr_ref, b1_ref, w2_ref, b2_ref,
             o_ref, vacc):
        i = pl.program_id(0)

        @pl.when(i == 0)
        def _():
            vacc[...] = jnp.zeros((1, DP), jnp.float32)

        dis = dr_ref[:, 0:1]
        acc = acc_ref[0] + acc_ref[1]
        h1 = dis * (acc + y_ref[...]) + b1_ref[...]
        r = jnp.maximum(h1, 0.0)
        sp = s_ref[0, :, 0:1] + s_ref[1, :, 0:1]
        sfull = dis * (sp + dis)
        rowid = i * ROWBLK + lax.broadcasted_iota(jnp.int32, (ROWBLK, 1), 0)
        sfull = jnp.where(rowid < N, sfull, 0.0)
        vacc[...] += jnp.sum(r * sfull, axis=0, keepdims=True)

        @pl.when(i == NROWBLK - 1)
        def _():
            v = vacc[...]
            pooled = jnp.dot(v, w2_ref[...],
                             preferred_element_type=jnp.float32)
            pooled = pooled * (1.0 / N) + b2_ref[...]
            laneid = lax.broadcasted_iota(jnp.int32, (1, CP), 1)
            valid = laneid < NCLS
            pm = jnp.where(valid, pooled, -1e30)
            m = jnp.max(pm, axis=1, keepdims=True)
            e = jnp.where(valid, jnp.exp(pooled - m), 0.0)
            lse = jnp.log(jnp.sum(e, axis=1, keepdims=True))
            o_ref[...] = pooled - m - lse

    return pl.pallas_call(
        body,
        grid=(NROWBLK,),
        in_specs=[
            pl.BlockSpec((NC, ROWBLK, DP), lambda i: (0, i, 0)),
            pl.BlockSpec((NC, ROWBLK, 16), lambda i: (0, i, 0)),
            pl.BlockSpec((ROWBLK, DP), lambda i: (i, 0)),
            pl.BlockSpec((ROWBLK, 16), lambda i: (i, 0)),
            pl.BlockSpec((1, DP), lambda i: (0, 0)),
            pl.BlockSpec((DP, CP), lambda i: (0, 0)),
            pl.BlockSpec((1, CP), lambda i: (0, 0)),
        ],
        out_specs=pl.BlockSpec((1, CP), lambda i: (0, 0)),
        out_shape=jax.ShapeDtypeStruct((1, CP), jnp.float32),
        scratch_shapes=[pltpu.VMEM((1, DP), jnp.float32)],
    )(accp, spart, ytab, disrows, b1p, W2p, b2p)


def kernel(x, edge_index, W1, b1, W2, b2):
    ei = edge_index.astype(jnp.int32)

    xp = jnp.pad(x.astype(jnp.float32), ((0, NPAD - N), (0, 0)))
    W1p = jnp.pad(W1.astype(jnp.float32), ((0, 0), (0, DP - D_HID)))
    b1p = jnp.pad(b1.astype(jnp.float32), (0, DP - D_HID)).reshape(1, DP)
    W2p = jnp.pad(W2.astype(jnp.float32), ((0, DP - D_HID), (0, CP - NCLS)))
    b2p = jnp.pad(b2.astype(jnp.float32), (0, CP - NCLS)).reshape(1, CP)

    ones16 = jnp.zeros((BLK, 16), jnp.float32).at[:, 0].set(1.0)
    zeros16 = jnp.zeros((RPW, 16), jnp.float32)
    zerosdp = jnp.zeros((RPW, DP), jnp.float32)

    degp = _deg_sc(ei, ones16, zeros16)
    h0 = _mm_tc(xp, W1p)
    ytab, disrows = _prep_tc(h0, degp)
    accp, spart = _agg_sc(ei, ytab, disrows, zerosdp, zeros16)
    out16 = _fin_tc(accp, spart, ytab, disrows, b1p, W2p, b2p)
    return out16[:, :NCLS]


# speedup vs baseline: 40.8275x; 1.0135x over previous
import dataclasses
import functools

import jax
import jax.numpy as jnp
from jax import lax
from jax.experimental import pallas as pl
from jax.experimental.pallas import tpu as pltpu
from jax.experimental.pallas import tpu_sc as plsc

N = 10000
E = 320000
D_IN = 128
D_HID = 100
DP = 128
NCLS = 10
CP = 16

NC = 2
NS = 16
NW = NC * NS
BLK = 128
NBLK_ALL = E // BLK
NBLK_LO = NBLK_ALL // NW
NBLK_XTRA = NBLK_ALL - NBLK_LO * NW
NPAD = 10240
RPW = NPAD // NS


def _worker_blocks(wid):
    nblk = jnp.where(wid < NBLK_XTRA, NBLK_LO + 1, NBLK_LO)
    base = wid * NBLK_LO + jnp.minimum(wid, NBLK_XTRA)
    return base, nblk

ROWBLK = 2048
NROWBLK = NPAD // ROWBLK

_MESH = plsc.VectorSubcoreMesh(core_axis_name="c", subcore_axis_name="s")
_SC_PARAMS = pltpu.CompilerParams(use_tc_tiling_on_sc=False)
_SC_PARAMS_NL = _SC_PARAMS
if "needs_layout_passes" in pltpu.CompilerParams.__dataclass_fields__:
    _SC_PARAMS_NL = dataclasses.replace(_SC_PARAMS, needs_layout_passes=False)


def _deg_sc(ei):
    @functools.partial(
        pl.kernel,
        mesh=_MESH,
        compiler_params=_SC_PARAMS_NL,
        out_type=jax.ShapeDtypeStruct((NC, NPAD, 16), jnp.float32),
        scratch_types=[
            pltpu.VMEM_SHARED((NS, NPAD), jnp.float32),
            pltpu.VMEM((NPAD,), jnp.float32),
            pltpu.VMEM((NS, RPW), jnp.float32),
            pltpu.VMEM((RPW,), jnp.float32),
            pltpu.VMEM((RPW, 16), jnp.float32),
            pltpu.VMEM((BLK,), jnp.int32),
            pltpu.VMEM((BLK,), jnp.int32),
            pltpu.SemaphoreType.DMA,
            pltpu.SemaphoreType.DMA,
        ],
    )
    def k(ei_hbm, deg_hbm, stag, hist_v, m_v, a_v, bcast_v, didx0, didx1,
          semI0, semI1):
        dst_hbm = ei_hbm.at[1]
        c = lax.axis_index("c")
        s = lax.axis_index("s")
        wid = c * NS + s
        base0, nblk = _worker_blocks(wid)
        didx = (didx0, didx1)
        semI = (semI0, semI1)

        pltpu.async_copy(dst_hbm.at[pl.ds(base0 * BLK, BLK)], didx0, semI0)

        zero16 = jnp.zeros((16,), jnp.float32)
        ones = jnp.ones((16,), jnp.float32)

        @pl.loop(0, NPAD // 16)
        def _(i):
            hist_v[pl.ds(i * 16, 16)] = zero16

        @pl.loop(0, (NBLK_LO + 2) // 2)
        def _(jj):
            j0 = jj * 2
            for p in (0, 1):
                j = j0 + p
                q = 1 - p

                @pl.when(j < nblk)
                def _():
                    pltpu.make_async_copy(dst_hbm.at[pl.ds(0, BLK)], didx[p],
                                          semI[p]).wait()

                    @pl.when(j < nblk - 1)
                    def _():
                        pltpu.async_copy(
                            dst_hbm.at[pl.ds((base0 + j + 1) * BLK, BLK)],
                            didx[q], semI[q])

                    for i in range(BLK // 16):
                        idxv = didx[p][pl.ds(i * 16, 16)]
                        plsc.addupdate_scatter(hist_v, [idxv], ones)

        pltpu.sync_copy(hist_v, stag.at[s])
        plsc.subcore_barrier()
        pltpu.sync_copy(stag.at[:, pl.ds(s * RPW, RPW)], m_v)

        @pl.loop(0, RPW // 16)
        def _(i):
            acc = m_v[0, pl.ds(i * 16, 16)]
            for r in range(1, NS):
                acc = acc + m_v[r, pl.ds(i * 16, 16)]
            a_v[pl.ds(i * 16, 16)] = acc

        @pl.loop(0, RPW)
        def _(r):
            bcast_v[r] = plsc.load_gather(a_v, [jnp.full((16,), r, jnp.int32)])

        pltpu.sync_copy(bcast_v, deg_hbm.at[c, pl.ds(s * RPW, RPW)])

    return k(ei)


def _agg_sc(ei, ytab, disrows, zdp, z16):
    @functools.partial(
        pl.kernel,
        mesh=_MESH,
        compiler_params=_SC_PARAMS,
        out_type=(
            jax.ShapeDtypeStruct((NC, NPAD, DP), jnp.float32),
            jax.ShapeDtypeStruct((NC, NPAD, 16), jnp.float32),
        ),
        scratch_types=[
            pltpu.VMEM_SHARED((NPAD, DP), jnp.float32),
            pltpu.VMEM_SHARED((NPAD, 16), jnp.float32),
            pltpu.VMEM((BLK, DP), jnp.float32),
            pltpu.VMEM((BLK, DP), jnp.float32),
            pltpu.VMEM((BLK, 16), jnp.float32),
            pltpu.VMEM((BLK, 16), jnp.float32),
            pltpu.VMEM((BLK,), jnp.int32),
            pltpu.VMEM((BLK,), jnp.int32),
            pltpu.VMEM((BLK,), jnp.int32),
            pltpu.VMEM((BLK,), jnp.int32),
            pltpu.SemaphoreType.DMA,
            pltpu.SemaphoreType.DMA,
            pltpu.SemaphoreType.DMA,
            pltpu.SemaphoreType.DMA,
            pltpu.SemaphoreType.DMA,
            pltpu.SemaphoreType.DMA,
        ],
    )
    def k(ei_hbm, y_hbm, dr_hbm, zdp_hbm, z16_hbm,
          acc_hbm, s_hbm, accsh, ssh, rows0, rows1, drows0, drows1,
          sidx0, sidx1, didx0, didx1, semI0, semI1, semG0, semG1,
          semD0, semD1):
        src_hbm = ei_hbm.at[0]
        dst_hbm = ei_hbm.at[1]
        c = lax.axis_index("c")
        s = lax.axis_index("s")
        wid = c * NS + s
        pltpu.sync_copy(zdp_hbm, accsh.at[pl.ds(s * RPW, RPW)])
        pltpu.sync_copy(z16_hbm, ssh.at[pl.ds(s * RPW, RPW)])
        plsc.subcore_barrier()
        base0, nblk = _worker_blocks(wid)
        rows = (rows0, rows1)
        drows = (drows0, drows1)
        sidx = (sidx0, sidx1)
        didx = (didx0, didx1)
        semI = (semI0, semI1)
        semG = (semG0, semG1)
        semD = (semD0, semD1)

        def idx_fetch(j, p):
            pltpu.async_copy(src_hbm.at[pl.ds((base0 + j) * BLK, BLK)],
                             sidx[p], semI[p])
            pltpu.async_copy(dst_hbm.at[pl.ds((base0 + j) * BLK, BLK)],
                             didx[p], semI[p])

        def idx_wait(p):
            pltpu.make_async_copy(src_hbm.at[pl.ds(0, BLK)], sidx[p],
                                  semI[p]).wait()
            pltpu.make_async_copy(src_hbm.at[pl.ds(0, BLK)], didx[p],
                                  semI[p]).wait()

        def gathers(p):
            pltpu.async_copy(y_hbm.at[sidx[p]], rows[p], semG[p])
            pltpu.async_copy(dr_hbm.at[didx[p]], drows[p], semD[p])

        def scatters(p):
            pltpu.make_async_copy(y_hbm.at[sidx[p]], rows[p], semG[p]).wait()
            pltpu.sync_copy(rows[p], accsh.at[didx[p]], add=True)
            pltpu.make_async_copy(dr_hbm.at[didx[p]], drows[p],
                                  semD[p]).wait()
            pltpu.sync_copy(drows[p], ssh.at[sidx[p]], add=True)

        idx_fetch(0, 0)
        idx_fetch(1, 1)
        idx_wait(0)
        gathers(0)

        @pl.loop(0, (NBLK_LO + 2) // 2)
        def _(jj):
            j0 = jj * 2
            for p in (0, 1):
                j = j0 + p
                q = 1 - p

                @pl.when(j < nblk)
                def _():
                    @pl.when(j < nblk - 1)
                    def _():
                        idx_wait(q)
                        gathers(q)

                    scatters(p)

                    @pl.when(j < nblk - 2)
                    def _():
                        idx_fetch(j + 2, p)

        plsc.subcore_barrier()
        pltpu.sync_copy(accsh.at[pl.ds(s * RPW, RPW)],
                        acc_hbm.at[c, pl.ds(s * RPW, RPW)])
        pltpu.sync_copy(ssh.at[pl.ds(s * RPW, RPW)],
                        s_hbm.at[c, pl.ds(s * RPW, RPW)])

    return k(ei, ytab, disrows, zdp, z16)


def _mm_tc(xp, W1p):
    def body(x_ref, w_ref, o_ref):
        o_ref[...] = jnp.dot(x_ref[...], w_ref[...],
                             preferred_element_type=jnp.float32)

    return pl.pallas_call(
        body,
        grid=(NROWBLK,),
        in_specs=[
            pl.BlockSpec((ROWBLK, D_IN), lambda i: (i, 0)),
            pl.BlockSpec((D_IN, DP), lambda i: (0, 0)),
        ],
        out_specs=pl.BlockSpec((ROWBLK, DP), lambda i: (i, 0)),
        out_shape=jax.ShapeDtypeStruct((NPAD, DP), jnp.float32),
    )(xp, W1p)


def _prep_tc(h0, degp):
    def body(h_ref, d_ref, y_ref, dr_ref):
        deg = d_ref[0, :, 0:1] + d_ref[1, :, 0:1] + 1.0
        dis = lax.rsqrt(deg)
        y_ref[...] = h_ref[...] * dis
        dr_ref[...] = jnp.broadcast_to(dis, (ROWBLK, 16))

    return pl.pallas_call(
        body,
        grid=(NROWBLK,),
        in_specs=[
            pl.BlockSpec((ROWBLK, DP), lambda i: (i, 0)),
            pl.BlockSpec((NC, ROWBLK, 16), lambda i: (0, i, 0)),
        ],
        out_specs=[
            pl.BlockSpec((ROWBLK, DP), lambda i: (i, 0)),
            pl.BlockSpec((ROWBLK, 16), lambda i: (i, 0)),
        ],
        out_shape=[
            jax.ShapeDtypeStruct((NPAD, DP), jnp.float32),
            jax.ShapeDtypeStruct((NPAD, 16), jnp.float32),
        ],
    )(h0, degp)


def _fin_tc(accp, spart, ytab, disrows, b1p, W2p, b2p):
    def body(acc_ref, s_ref, y_ref, dr_ref, b1_ref, w2_ref, b2_ref,
             o_ref, vacc):
        i = pl.program_id(0)

        @pl.when(i == 0)
        def _():
            vacc[...] = jnp.zeros((1, DP), jnp.float32)

        dis = dr_ref[:, 0:1]
        acc = acc_ref[0] + acc_ref[1]
        h1 = dis * (acc + y_ref[...]) + b1_ref[...]
        r = jnp.maximum(h1, 0.0)
        sp = s_ref[0, :, 0:1] + s_ref[1, :, 0:1]
        sfull = dis * (sp + dis)
        rowid = i * ROWBLK + lax.broadcasted_iota(jnp.int32, (ROWBLK, 1), 0)
        sfull = jnp.where(rowid < N, sfull, 0.0)
        vacc[...] += jnp.sum(r * sfull, axis=0, keepdims=True)

        @pl.when(i == NROWBLK - 1)
        def _():
            v = vacc[...]
            pooled = jnp.dot(v, w2_ref[...],
                             preferred_element_type=jnp.float32)
            pooled = pooled * (1.0 / N) + b2_ref[...]
            laneid = lax.broadcasted_iota(jnp.int32, (1, CP), 1)
            valid = laneid < NCLS
            pm = jnp.where(valid, pooled, -1e30)
            m = jnp.max(pm, axis=1, keepdims=True)
            e = jnp.where(valid, jnp.exp(pooled - m), 0.0)
            lse = jnp.log(jnp.sum(e, axis=1, keepdims=True))
            o_ref[...] = pooled - m - lse

    return pl.pallas_call(
        body,
        grid=(NROWBLK,),
        in_specs=[
            pl.BlockSpec((NC, ROWBLK, DP), lambda i: (0, i, 0)),
            pl.BlockSpec((NC, ROWBLK, 16), lambda i: (0, i, 0)),
            pl.BlockSpec((ROWBLK, DP), lambda i: (i, 0)),
            pl.BlockSpec((ROWBLK, 16), lambda i: (i, 0)),
            pl.BlockSpec((1, DP), lambda i: (0, 0)),
            pl.BlockSpec((DP, CP), lambda i: (0, 0)),
            pl.BlockSpec((1, CP), lambda i: (0, 0)),
        ],
        out_specs=pl.BlockSpec((1, CP), lambda i: (0, 0)),
        out_shape=jax.ShapeDtypeStruct((1, CP), jnp.float32),
        scratch_shapes=[pltpu.VMEM((1, DP), jnp.float32)],
    )(accp, spart, ytab, disrows, b1p, W2p, b2p)


def kernel(x, edge_index, W1, b1, W2, b2):
    ei = edge_index.astype(jnp.int32)

    xp = jnp.pad(x.astype(jnp.float32), ((0, NPAD - N), (0, 0)))
    W1p = jnp.pad(W1.astype(jnp.float32), ((0, 0), (0, DP - D_HID)))
    b1p = jnp.pad(b1.astype(jnp.float32), (0, DP - D_HID)).reshape(1, DP)
    W2p = jnp.pad(W2.astype(jnp.float32), ((0, DP - D_HID), (0, CP - NCLS)))
    b2p = jnp.pad(b2.astype(jnp.float32), (0, CP - NCLS)).reshape(1, CP)

    zeros16 = jnp.zeros((RPW, 16), jnp.float32)
    zerosdp = jnp.zeros((RPW, DP), jnp.float32)

    degp = _deg_sc(ei)
    h0 = _mm_tc(xp, W1p)
    ytab, disrows = _prep_tc(h0, degp)
    accp, spart = _agg_sc(ei, ytab, disrows, zerosdp, zeros16)
    out16 = _fin_tc(accp, spart, ytab, disrows, b1p, W2p, b2p)
    return out16[:, :NCLS]


# speedup vs baseline: 47.5494x; 1.1646x over previous
import dataclasses
import functools

import jax
import jax.numpy as jnp
from jax import lax
from jax.experimental import pallas as pl
from jax.experimental.pallas import tpu as pltpu
from jax.experimental.pallas import tpu_sc as plsc

N = 10000
E = 320000
D_IN = 128
D_HID = 100
DP = 128
NCLS = 10
CP = 16

NC = 2
NS = 16
NW = NC * NS
BLK = 128
NBLK_ALL = E // BLK
NBLK_LO = NBLK_ALL // NW
NBLK_XTRA = NBLK_ALL - NBLK_LO * NW
DEG_GRP = 2000
DEG_NGRP = (E // NW) // DEG_GRP
NPAD = 10240
RPW = NPAD // NS


def _worker_blocks(wid):
    nblk = jnp.where(wid < NBLK_XTRA, NBLK_LO + 1, NBLK_LO)
    base = wid * NBLK_LO + jnp.minimum(wid, NBLK_XTRA)
    return base, nblk

ROWBLK = 2048
NROWBLK = NPAD // ROWBLK

_MESH = plsc.VectorSubcoreMesh(core_axis_name="c", subcore_axis_name="s")
_SC_PARAMS = pltpu.CompilerParams(use_tc_tiling_on_sc=False)
_SC_PARAMS_NL = _SC_PARAMS
if "needs_layout_passes" in pltpu.CompilerParams.__dataclass_fields__:
    _SC_PARAMS_NL = dataclasses.replace(_SC_PARAMS, needs_layout_passes=False)


def _deg_sc(ei):
    @functools.partial(
        pl.kernel,
        mesh=_MESH,
        compiler_params=_SC_PARAMS_NL,
        out_type=jax.ShapeDtypeStruct((NC, NPAD, 16), jnp.float32),
        scratch_types=[
            pltpu.VMEM_SHARED((NS, NPAD), jnp.float32),
            pltpu.VMEM((NPAD,), jnp.float32),
            pltpu.VMEM((NS, RPW), jnp.float32),
            pltpu.VMEM((RPW,), jnp.float32),
            pltpu.VMEM((RPW, 16), jnp.float32),
            pltpu.VMEM((DEG_GRP,), jnp.int32),
            pltpu.VMEM((DEG_GRP,), jnp.int32),
            pltpu.SemaphoreType.DMA,
            pltpu.SemaphoreType.DMA,
        ],
    )
    def k(ei_hbm, deg_hbm, stag, hist_v, m_v, a_v, bcast_v, didx0, didx1,
          semI0, semI1):
        dst_hbm = ei_hbm.at[1]
        c = lax.axis_index("c")
        s = lax.axis_index("s")
        wid = c * NS + s
        base0 = wid * (E // NW)
        didx = (didx0, didx1)
        semI = (semI0, semI1)

        pltpu.async_copy(dst_hbm.at[pl.ds(base0, DEG_GRP)], didx0, semI0)

        zero16 = jnp.zeros((16,), jnp.float32)
        ones = jnp.ones((16,), jnp.float32)

        @pl.loop(0, NPAD // 16)
        def _(i):
            hist_v[pl.ds(i * 16, 16)] = zero16

        for g in range(DEG_NGRP):
            p = g % 2
            q = 1 - p
            pltpu.make_async_copy(dst_hbm.at[pl.ds(0, DEG_GRP)], didx[p],
                                  semI[p]).wait()
            if g < DEG_NGRP - 1:
                pltpu.async_copy(
                    dst_hbm.at[pl.ds(base0 + (g + 1) * DEG_GRP, DEG_GRP)],
                    didx[q], semI[q])

            @pl.loop(0, DEG_GRP // 16)
            def _(i):
                idxv = didx[p][pl.ds(i * 16, 16)]
                plsc.addupdate_scatter(hist_v, [idxv], ones)

        pltpu.sync_copy(hist_v, stag.at[s])
        plsc.subcore_barrier()
        pltpu.sync_copy(stag.at[:, pl.ds(s * RPW, RPW)], m_v)

        @pl.loop(0, RPW // 16)
        def _(i):
            acc = m_v[0, pl.ds(i * 16, 16)]
            for r in range(1, NS):
                acc = acc + m_v[r, pl.ds(i * 16, 16)]
            a_v[pl.ds(i * 16, 16)] = acc
            rowidx = i * 16 + lax.iota(jnp.int32, 16)
            plsc.store_scatter(bcast_v, [rowidx, jnp.zeros((16,), jnp.int32)],
                               acc)

        pltpu.sync_copy(bcast_v, deg_hbm.at[c, pl.ds(s * RPW, RPW)])

    return k(ei)


def _agg_sc(ei, ytab, disrows, zdp, z16):
    @functools.partial(
        pl.kernel,
        mesh=_MESH,
        compiler_params=_SC_PARAMS,
        out_type=(
            jax.ShapeDtypeStruct((NC, NPAD, DP), jnp.float32),
            jax.ShapeDtypeStruct((NC, NPAD, 16), jnp.float32),
        ),
        scratch_types=[
            pltpu.VMEM_SHARED((NPAD, DP), jnp.float32),
            pltpu.VMEM_SHARED((NPAD, 16), jnp.float32),
            pltpu.VMEM((BLK, DP), jnp.float32),
            pltpu.VMEM((BLK, DP), jnp.float32),
            pltpu.VMEM((BLK, 16), jnp.float32),
            pltpu.VMEM((BLK, 16), jnp.float32),
            pltpu.VMEM((BLK,), jnp.int32),
            pltpu.VMEM((BLK,), jnp.int32),
            pltpu.VMEM((BLK,), jnp.int32),
            pltpu.VMEM((BLK,), jnp.int32),
            pltpu.SemaphoreType.DMA,
            pltpu.SemaphoreType.DMA,
            pltpu.SemaphoreType.DMA,
            pltpu.SemaphoreType.DMA,
            pltpu.SemaphoreType.DMA,
            pltpu.SemaphoreType.DMA,
        ],
    )
    def k(ei_hbm, y_hbm, dr_hbm, zdp_hbm, z16_hbm,
          acc_hbm, s_hbm, accsh, ssh, rows0, rows1, drows0, drows1,
          sidx0, sidx1, didx0, didx1, semI0, semI1, semG0, semG1,
          semD0, semD1):
        src_hbm = ei_hbm.at[0]
        dst_hbm = ei_hbm.at[1]
        c = lax.axis_index("c")
        s = lax.axis_index("s")
        wid = c * NS + s
        pltpu.sync_copy(zdp_hbm, accsh.at[pl.ds(s * RPW, RPW)])
        pltpu.sync_copy(z16_hbm, ssh.at[pl.ds(s * RPW, RPW)])
        plsc.subcore_barrier()
        base0, nblk = _worker_blocks(wid)
        rows = (rows0, rows1)
        drows = (drows0, drows1)
        sidx = (sidx0, sidx1)
        didx = (didx0, didx1)
        semI = (semI0, semI1)
        semG = (semG0, semG1)
        semD = (semD0, semD1)

        def idx_fetch(j, p):
            pltpu.async_copy(src_hbm.at[pl.ds((base0 + j) * BLK, BLK)],
                             sidx[p], semI[p])
            pltpu.async_copy(dst_hbm.at[pl.ds((base0 + j) * BLK, BLK)],
                             didx[p], semI[p])

        def idx_wait(p):
            pltpu.make_async_copy(src_hbm.at[pl.ds(0, BLK)], sidx[p],
                                  semI[p]).wait()
            pltpu.make_async_copy(src_hbm.at[pl.ds(0, BLK)], didx[p],
                                  semI[p]).wait()

        def gathers(p):
            pltpu.async_copy(y_hbm.at[sidx[p]], rows[p], semG[p])
            pltpu.async_copy(dr_hbm.at[didx[p]], drows[p], semD[p])

        def scatters(p):
            pltpu.make_async_copy(y_hbm.at[sidx[p]], rows[p], semG[p]).wait()
            pltpu.sync_copy(rows[p], accsh.at[didx[p]], add=True)
            pltpu.make_async_copy(dr_hbm.at[didx[p]], drows[p],
                                  semD[p]).wait()
            pltpu.sync_copy(drows[p], ssh.at[sidx[p]], add=True)

        idx_fetch(0, 0)
        idx_fetch(1, 1)
        idx_wait(0)
        gathers(0)

        @pl.loop(0, (NBLK_LO + 2) // 2)
        def _(jj):
            j0 = jj * 2
            for p in (0, 1):
                j = j0 + p
                q = 1 - p

                @pl.when(j < nblk)
                def _():
                    @pl.when(j < nblk - 1)
                    def _():
                        idx_wait(q)
                        gathers(q)

                    scatters(p)

                    @pl.when(j < nblk - 2)
                    def _():
                        idx_fetch(j + 2, p)

        plsc.subcore_barrier()
        pltpu.sync_copy(accsh.at[pl.ds(s * RPW, RPW)],
                        acc_hbm.at[c, pl.ds(s * RPW, RPW)])
        pltpu.sync_copy(ssh.at[pl.ds(s * RPW, RPW)],
                        s_hbm.at[c, pl.ds(s * RPW, RPW)])

    return k(ei, ytab, disrows, zdp, z16)


def _mm_tc(xp, W1p):
    def body(x_ref, w_ref, o_ref):
        o_ref[...] = jnp.dot(x_ref[...], w_ref[...],
                             preferred_element_type=jnp.float32)

    return pl.pallas_call(
        body,
        grid=(NROWBLK,),
        in_specs=[
            pl.BlockSpec((ROWBLK, D_IN), lambda i: (i, 0)),
            pl.BlockSpec((D_IN, DP), lambda i: (0, 0)),
        ],
        out_specs=pl.BlockSpec((ROWBLK, DP), lambda i: (i, 0)),
        out_shape=jax.ShapeDtypeStruct((NPAD, DP), jnp.float32),
    )(xp, W1p)


def _prep_tc(h0, degp):
    def body(h_ref, d_ref, y_ref, dr_ref):
        deg = d_ref[0, :, 0:1] + d_ref[1, :, 0:1] + 1.0
        dis = lax.rsqrt(deg)
        y_ref[...] = h_ref[...] * dis
        dr_ref[...] = jnp.broadcast_to(dis, (ROWBLK, 16))

    return pl.pallas_call(
        body,
        grid=(NROWBLK,),
        in_specs=[
            pl.BlockSpec((ROWBLK, DP), lambda i: (i, 0)),
            pl.BlockSpec((NC, ROWBLK, 16), lambda i: (0, i, 0)),
        ],
        out_specs=[
            pl.BlockSpec((ROWBLK, DP), lambda i: (i, 0)),
            pl.BlockSpec((ROWBLK, 16), lambda i: (i, 0)),
        ],
        out_shape=[
            jax.ShapeDtypeStruct((NPAD, DP), jnp.float32),
            jax.ShapeDtypeStruct((NPAD, 16), jnp.float32),
        ],
    )(h0, degp)


def _fin_tc(accp, spart, ytab, disrows, b1p, W2p, b2p):
    def body(acc_ref, s_ref, y_ref, dr_ref, b1_ref, w2_ref, b2_ref,
             o_ref, vacc):
        i = pl.program_id(0)

        @pl.when(i == 0)
        def _():
            vacc[...] = jnp.zeros((1, DP), jnp.float32)

        dis = dr_ref[:, 0:1]
        acc = acc_ref[0] + acc_ref[1]
        h1 = dis * (acc + y_ref[...]) + b1_ref[...]
        r = jnp.maximum(h1, 0.0)
        sp = s_ref[0, :, 0:1] + s_ref[1, :, 0:1]
        sfull = dis * (sp + dis)
        rowid = i * ROWBLK + lax.broadcasted_iota(jnp.int32, (ROWBLK, 1), 0)
        sfull = jnp.where(rowid < N, sfull, 0.0)
        vacc[...] += jnp.sum(r * sfull, axis=0, keepdims=True)

        @pl.when(i == NROWBLK - 1)
        def _():
            v = vacc[...]
            pooled = jnp.dot(v, w2_ref[...],
                             preferred_element_type=jnp.float32)
            pooled = pooled * (1.0 / N) + b2_ref[...]
            laneid = lax.broadcasted_iota(jnp.int32, (1, CP), 1)
            valid = laneid < NCLS
            pm = jnp.where(valid, pooled, -1e30)
            m = jnp.max(pm, axis=1, keepdims=True)
            e = jnp.where(valid, jnp.exp(pooled - m), 0.0)
            lse = jnp.log(jnp.sum(e, axis=1, keepdims=True))
            o_ref[...] = pooled - m - lse

    return pl.pallas_call(
        body,
        grid=(NROWBLK,),
        in_specs=[
            pl.BlockSpec((NC, ROWBLK, DP), lambda i: (0, i, 0)),
            pl.BlockSpec((NC, ROWBLK, 16), lambda i: (0, i, 0)),
            pl.BlockSpec((ROWBLK, DP), lambda i: (i, 0)),
            pl.BlockSpec((ROWBLK, 16), lambda i: (i, 0)),
            pl.BlockSpec((1, DP), lambda i: (0, 0)),
            pl.BlockSpec((DP, CP), lambda i: (0, 0)),
            pl.BlockSpec((1, CP), lambda i: (0, 0)),
        ],
        out_specs=pl.BlockSpec((1, CP), lambda i: (0, 0)),
        out_shape=jax.ShapeDtypeStruct((1, CP), jnp.float32),
        scratch_shapes=[pltpu.VMEM((1, DP), jnp.float32)],
    )(accp, spart, ytab, disrows, b1p, W2p, b2p)


def kernel(x, edge_index, W1, b1, W2, b2):
    ei = edge_index.astype(jnp.int32)

    xp = jnp.pad(x.astype(jnp.float32), ((0, NPAD - N), (0, 0)))
    W1p = jnp.pad(W1.astype(jnp.float32), ((0, 0), (0, DP - D_HID)))
    b1p = jnp.pad(b1.astype(jnp.float32), (0, DP - D_HID)).reshape(1, DP)
    W2p = jnp.pad(W2.astype(jnp.float32), ((0, DP - D_HID), (0, CP - NCLS)))
    b2p = jnp.pad(b2.astype(jnp.float32), (0, CP - NCLS)).reshape(1, CP)

    zeros16 = jnp.zeros((RPW, 16), jnp.float32)
    zerosdp = jnp.zeros((RPW, DP), jnp.float32)

    degp = _deg_sc(ei)
    h0 = _mm_tc(xp, W1p)
    ytab, disrows = _prep_tc(h0, degp)
    accp, spart = _agg_sc(ei, ytab, disrows, zerosdp, zeros16)
    out16 = _fin_tc(accp, spart, ytab, disrows, b1p, W2p, b2p)
    return out16[:, :NCLS]


# speedup vs baseline: 48.7626x; 1.0255x over previous
import dataclasses
import functools

import jax
import jax.numpy as jnp
from jax import lax
from jax.experimental import pallas as pl
from jax.experimental.pallas import tpu as pltpu
from jax.experimental.pallas import tpu_sc as plsc

N = 10000
E = 320000
D_IN = 128
D_HID = 100
DP = 128
NCLS = 10
CP = 16

NC = 2
NS = 16
NW = NC * NS
BLK = 128
NBLK_ALL = E // BLK
NBLK_LO = NBLK_ALL // NW
NBLK_XTRA = NBLK_ALL - NBLK_LO * NW
DEG_GRP = 2000
DEG_NGRP = (E // NW) // DEG_GRP
NPAD = 10240
RPW = NPAD // NS


def _worker_blocks(wid):
    nblk = jnp.where(wid < NBLK_XTRA, NBLK_LO + 1, NBLK_LO)
    base = wid * NBLK_LO + jnp.minimum(wid, NBLK_XTRA)
    return base, nblk

ROWBLK = 2048
NROWBLK = NPAD // ROWBLK

_MESH = plsc.VectorSubcoreMesh(core_axis_name="c", subcore_axis_name="s")
_SC_PARAMS = pltpu.CompilerParams(use_tc_tiling_on_sc=False)
_SC_PARAMS_NL = _SC_PARAMS
if "needs_layout_passes" in pltpu.CompilerParams.__dataclass_fields__:
    _SC_PARAMS_NL = dataclasses.replace(_SC_PARAMS, needs_layout_passes=False)


def _deg_sc(ei):
    @functools.partial(
        pl.kernel,
        mesh=_MESH,
        compiler_params=_SC_PARAMS_NL,
        out_type=jax.ShapeDtypeStruct((NC, NPAD, 16), jnp.float32),
        scratch_types=[
            pltpu.VMEM_SHARED((NS, NPAD), jnp.float32),
            pltpu.VMEM((NPAD,), jnp.float32),
            pltpu.VMEM((NS, RPW), jnp.float32),
            pltpu.VMEM((RPW,), jnp.float32),
            pltpu.VMEM((RPW, 16), jnp.float32),
            pltpu.VMEM((DEG_GRP,), jnp.int32),
            pltpu.VMEM((DEG_GRP,), jnp.int32),
            pltpu.SemaphoreType.DMA,
            pltpu.SemaphoreType.DMA,
        ],
    )
    def k(ei_hbm, deg_hbm, stag, hist_v, m_v, a_v, bcast_v, didx0, didx1,
          semI0, semI1):
        dst_hbm = ei_hbm.at[1]
        c = lax.axis_index("c")
        s = lax.axis_index("s")
        wid = c * NS + s
        base0 = wid * (E // NW)
        didx = (didx0, didx1)
        semI = (semI0, semI1)

        pltpu.async_copy(dst_hbm.at[pl.ds(base0, DEG_GRP)], didx0, semI0)

        zero16 = jnp.zeros((16,), jnp.float32)
        ones = jnp.ones((16,), jnp.float32)

        @pl.loop(0, NPAD // 16)
        def _(i):
            hist_v[pl.ds(i * 16, 16)] = zero16

        for g in range(DEG_NGRP):
            p = g % 2
            q = 1 - p
            pltpu.make_async_copy(dst_hbm.at[pl.ds(0, DEG_GRP)], didx[p],
                                  semI[p]).wait()
            if g < DEG_NGRP - 1:
                pltpu.async_copy(
                    dst_hbm.at[pl.ds(base0 + (g + 1) * DEG_GRP, DEG_GRP)],
                    didx[q], semI[q])

            @pl.loop(0, DEG_GRP // 16)
            def _(i):
                idxv = didx[p][pl.ds(i * 16, 16)]
                plsc.addupdate_scatter(hist_v, [idxv], ones)

        pltpu.sync_copy(hist_v, stag.at[s])
        plsc.subcore_barrier()
        pltpu.sync_copy(stag.at[:, pl.ds(s * RPW, RPW)], m_v)

        @pl.loop(0, RPW // 16)
        def _(i):
            acc = m_v[0, pl.ds(i * 16, 16)]
            for r in range(1, NS):
                acc = acc + m_v[r, pl.ds(i * 16, 16)]
            a_v[pl.ds(i * 16, 16)] = acc
            rowidx = i * 16 + lax.iota(jnp.int32, 16)
            plsc.store_scatter(bcast_v, [rowidx, jnp.zeros((16,), jnp.int32)],
                               acc)

        pltpu.sync_copy(bcast_v, deg_hbm.at[c, pl.ds(s * RPW, RPW)])

    return k(ei)


def _agg_sc(ei, ytab, disrows):
    @functools.partial(
        pl.kernel,
        mesh=_MESH,
        compiler_params=_SC_PARAMS,
        out_type=(
            jax.ShapeDtypeStruct((NC, NPAD, DP), jnp.float32),
            jax.ShapeDtypeStruct((NC, NPAD, 16), jnp.float32),
        ),
        scratch_types=[
            pltpu.VMEM_SHARED((NPAD, DP), jnp.float32),
            pltpu.VMEM_SHARED((NPAD, 16), jnp.float32),
            pltpu.VMEM((BLK, DP), jnp.float32),
            pltpu.VMEM((BLK, DP), jnp.float32),
            pltpu.VMEM((BLK, 16), jnp.float32),
            pltpu.VMEM((BLK, 16), jnp.float32),
            pltpu.VMEM((BLK,), jnp.int32),
            pltpu.VMEM((BLK,), jnp.int32),
            pltpu.VMEM((BLK,), jnp.int32),
            pltpu.VMEM((BLK,), jnp.int32),
            pltpu.SemaphoreType.DMA,
            pltpu.SemaphoreType.DMA,
            pltpu.SemaphoreType.DMA,
            pltpu.SemaphoreType.DMA,
            pltpu.SemaphoreType.DMA,
            pltpu.SemaphoreType.DMA,
        ],
    )
    def k(ei_hbm, y_hbm, dr_hbm,
          acc_hbm, s_hbm, accsh, ssh, rows0, rows1, drows0, drows1,
          sidx0, sidx1, didx0, didx1, semI0, semI1, semG0, semG1,
          semD0, semD1):
        src_hbm = ei_hbm.at[0]
        dst_hbm = ei_hbm.at[1]
        c = lax.axis_index("c")
        s = lax.axis_index("s")
        wid = c * NS + s
        zero16 = jnp.zeros((16,), jnp.float32)

        @pl.loop(0, BLK)
        def _(r):
            for l in range(DP // 16):
                rows0[r, pl.ds(l * 16, 16)] = zero16
            drows0[r] = zero16

        for kk in range(RPW // BLK):
            pltpu.sync_copy(rows0, accsh.at[pl.ds(s * RPW + kk * BLK, BLK)])
            pltpu.sync_copy(drows0, ssh.at[pl.ds(s * RPW + kk * BLK, BLK)])
        plsc.subcore_barrier()
        base0, nblk = _worker_blocks(wid)
        rows = (rows0, rows1)
        drows = (drows0, drows1)
        sidx = (sidx0, sidx1)
        didx = (didx0, didx1)
        semI = (semI0, semI1)
        semG = (semG0, semG1)
        semD = (semD0, semD1)

        def idx_fetch(j, p):
            pltpu.async_copy(src_hbm.at[pl.ds((base0 + j) * BLK, BLK)],
                             sidx[p], semI[p])
            pltpu.async_copy(dst_hbm.at[pl.ds((base0 + j) * BLK, BLK)],
                             didx[p], semI[p])

        def idx_wait(p):
            pltpu.make_async_copy(src_hbm.at[pl.ds(0, BLK)], sidx[p],
                                  semI[p]).wait()
            pltpu.make_async_copy(src_hbm.at[pl.ds(0, BLK)], didx[p],
                                  semI[p]).wait()

        def gathers(p):
            pltpu.async_copy(y_hbm.at[sidx[p]], rows[p], semG[p])
            pltpu.async_copy(dr_hbm.at[didx[p]], drows[p], semD[p])

        def scatters(p):
            pltpu.make_async_copy(y_hbm.at[sidx[p]], rows[p], semG[p]).wait()
            pltpu.sync_copy(rows[p], accsh.at[didx[p]], add=True)
            pltpu.make_async_copy(dr_hbm.at[didx[p]], drows[p],
                                  semD[p]).wait()
            pltpu.sync_copy(drows[p], ssh.at[sidx[p]], add=True)

        idx_fetch(0, 0)
        idx_fetch(1, 1)
        idx_wait(0)
        gathers(0)

        @pl.loop(0, (NBLK_LO + 2) // 2)
        def _(jj):
            j0 = jj * 2
            for p in (0, 1):
                j = j0 + p
                q = 1 - p

                @pl.when(j < nblk)
                def _():
                    @pl.when(j < nblk - 1)
                    def _():
                        idx_wait(q)
                        gathers(q)

                    scatters(p)

                    @pl.when(j < nblk - 2)
                    def _():
                        idx_fetch(j + 2, p)

        plsc.subcore_barrier()
        pltpu.sync_copy(accsh.at[pl.ds(s * RPW, RPW)],
                        acc_hbm.at[c, pl.ds(s * RPW, RPW)])
        pltpu.sync_copy(ssh.at[pl.ds(s * RPW, RPW)],
                        s_hbm.at[c, pl.ds(s * RPW, RPW)])

    return k(ei, ytab, disrows)


def _mm_tc(xp, W1p):
    def body(x_ref, w_ref, o_ref):
        o_ref[...] = jnp.dot(x_ref[...], w_ref[...],
                             preferred_element_type=jnp.float32)

    return pl.pallas_call(
        body,
        grid=(NROWBLK,),
        in_specs=[
            pl.BlockSpec((ROWBLK, D_IN), lambda i: (i, 0)),
            pl.BlockSpec((D_IN, DP), lambda i: (0, 0)),
        ],
        out_specs=pl.BlockSpec((ROWBLK, DP), lambda i: (i, 0)),
        out_shape=jax.ShapeDtypeStruct((NPAD, DP), jnp.float32),
    )(xp, W1p)


def _prep_tc(h0, degp):
    def body(h_ref, d_ref, y_ref, dr_ref):
        deg = d_ref[0, :, 0:1] + d_ref[1, :, 0:1] + 1.0
        dis = lax.rsqrt(deg)
        y_ref[...] = h_ref[...] * dis
        dr_ref[...] = jnp.broadcast_to(dis, (ROWBLK, 16))

    return pl.pallas_call(
        body,
        grid=(NROWBLK,),
        in_specs=[
            pl.BlockSpec((ROWBLK, DP), lambda i: (i, 0)),
            pl.BlockSpec((NC, ROWBLK, 16), lambda i: (0, i, 0)),
        ],
        out_specs=[
            pl.BlockSpec((ROWBLK, DP), lambda i: (i, 0)),
            pl.BlockSpec((ROWBLK, 16), lambda i: (i, 0)),
        ],
        out_shape=[
            jax.ShapeDtypeStruct((NPAD, DP), jnp.float32),
            jax.ShapeDtypeStruct((NPAD, 16), jnp.float32),
        ],
    )(h0, degp)


def _fin_tc(accp, spart, ytab, disrows, b1p, W2p, b2p):
    def body(acc_ref, s_ref, y_ref, dr_ref, b1_ref, w2_ref, b2_ref,
             o_ref, vacc):
        i = pl.program_id(0)

        @pl.when(i == 0)
        def _():
            vacc[...] = jnp.zeros((1, DP), jnp.float32)

        dis = dr_ref[:, 0:1]
        acc = acc_ref[0] + acc_ref[1]
        h1 = dis * (acc + y_ref[...]) + b1_ref[...]
        r = jnp.maximum(h1, 0.0)
        sp = s_ref[0, :, 0:1] + s_ref[1, :, 0:1]
        sfull = dis * (sp + dis)
        rowid = i * ROWBLK + lax.broadcasted_iota(jnp.int32, (ROWBLK, 1), 0)
        sfull = jnp.where(rowid < N, sfull, 0.0)
        vacc[...] += jnp.sum(r * sfull, axis=0, keepdims=True)

        @pl.when(i == NROWBLK - 1)
        def _():
            v = vacc[...]
            pooled = jnp.dot(v, w2_ref[...],
                             preferred_element_type=jnp.float32)
            pooled = pooled * (1.0 / N) + b2_ref[...]
            laneid = lax.broadcasted_iota(jnp.int32, (1, CP), 1)
            valid = laneid < NCLS
            pm = jnp.where(valid, pooled, -1e30)
            m = jnp.max(pm, axis=1, keepdims=True)
            e = jnp.where(valid, jnp.exp(pooled - m), 0.0)
            lse = jnp.log(jnp.sum(e, axis=1, keepdims=True))
            o_ref[...] = pooled - m - lse

    return pl.pallas_call(
        body,
        grid=(NROWBLK,),
        in_specs=[
            pl.BlockSpec((NC, ROWBLK, DP), lambda i: (0, i, 0)),
            pl.BlockSpec((NC, ROWBLK, 16), lambda i: (0, i, 0)),
            pl.BlockSpec((ROWBLK, DP), lambda i: (i, 0)),
            pl.BlockSpec((ROWBLK, 16), lambda i: (i, 0)),
            pl.BlockSpec((1, DP), lambda i: (0, 0)),
            pl.BlockSpec((DP, CP), lambda i: (0, 0)),
            pl.BlockSpec((1, CP), lambda i: (0, 0)),
        ],
        out_specs=pl.BlockSpec((1, CP), lambda i: (0, 0)),
        out_shape=jax.ShapeDtypeStruct((1, CP), jnp.float32),
        scratch_shapes=[pltpu.VMEM((1, DP), jnp.float32)],
    )(accp, spart, ytab, disrows, b1p, W2p, b2p)


def kernel(x, edge_index, W1, b1, W2, b2):
    ei = edge_index.astype(jnp.int32)

    xp = jnp.pad(x.astype(jnp.float32), ((0, NPAD - N), (0, 0)))
    W1p = jnp.pad(W1.astype(jnp.float32), ((0, 0), (0, DP - D_HID)))
    b1p = jnp.pad(b1.astype(jnp.float32), (0, DP - D_HID)).reshape(1, DP)
    W2p = jnp.pad(W2.astype(jnp.float32), ((0, DP - D_HID), (0, CP - NCLS)))
    b2p = jnp.pad(b2.astype(jnp.float32), (0, CP - NCLS)).reshape(1, CP)

    degp = _deg_sc(ei)
    h0 = _mm_tc(xp, W1p)
    ytab, disrows = _prep_tc(h0, degp)
    accp, spart = _agg_sc(ei, ytab, disrows)
    out16 = _fin_tc(accp, spart, ytab, disrows, b1p, W2p, b2p)
    return out16[:, :NCLS]
